# Initial kernel scaffold; baseline (speedup 1.0000x reference)
import jax
import jax.numpy as jnp
from jax.experimental import pallas as pl


def kernel(coords, hidden, edges, W1, b1, W2, b2, Wc1, bc1, Wc2, Wh1, bh1, Wh2, bh2):
    raise NotImplementedError("write your pallas kernel here")


# speedup vs baseline: 4.3530x; 4.3530x over previous
import functools

import jax
import jax.numpy as jnp
from jax import lax
from jax.experimental import pallas as pl
from jax.experimental.pallas import tpu as pltpu
from jax.experimental.pallas import tpu_sc as plsc

F32 = jnp.float32
I32 = jnp.int32

NC = 2
NS = 16
NW = NC * NS

SUB = 80
KSUB = 5
SCH = SUB * KSUB


def _iota16():
    return lax.iota(I32, 16)


def _precompute_body(h_ref, wa_ref, wb_ref, p_ref, q_ref):
    h = h_ref[...]
    p_ref[...] = jnp.dot(h, wa_ref[...], preferred_element_type=F32)
    q_ref[...] = jnp.dot(h, wb_ref[...], preferred_element_type=F32)


def _precompute(hidden, W1a, W1b, blk):
    n, hdim = hidden.shape
    m = W1a.shape[1]
    return pl.pallas_call(
        _precompute_body,
        grid=(n // blk,),
        in_specs=[
            pl.BlockSpec((blk, hdim), lambda i: (i, 0)),
            pl.BlockSpec((hdim, m), lambda i: (0, 0)),
            pl.BlockSpec((hdim, m), lambda i: (0, 0)),
        ],
        out_specs=[
            pl.BlockSpec((blk, m), lambda i: (i, 0)),
            pl.BlockSpec((blk, m), lambda i: (i, 0)),
        ],
        out_shape=[
            jax.ShapeDtypeStruct((n, m), F32),
            jax.ShapeDtypeStruct((n, m), F32),
        ],
    )(hidden, W1a, W1b)


def _make_gather(E, M):
    T = E // NW
    n_super = T // SCH
    mesh = plsc.VectorSubcoreMesh(core_axis_name="c", subcore_axis_name="s")

    @functools.partial(
        pl.kernel,
        mesh=mesh,
        out_type=[
            jax.ShapeDtypeStruct((E, M), F32),
            jax.ShapeDtypeStruct((E, M), F32),
        ],
        scratch_types=[
            pltpu.VMEM((SCH,), I32),
            pltpu.VMEM((SCH,), I32),
            pltpu.VMEM((SCH, M), F32),
            pltpu.VMEM((SCH, M), F32),
            pltpu.SemaphoreType.DMA,
        ],
    )
    def gather_kernel(p_hbm, q_hbm, e0_hbm, e1_hbm, ga_hbm, gb_hbm,
                      idx0, idx1, bufa, bufb, sem):
        wid = lax.axis_index("s") * NC + lax.axis_index("c")

        def body(j, _):
            base = wid * T + j * SCH
            pltpu.sync_copy(e0_hbm.at[pl.ds(base, SCH)], idx0)
            pltpu.sync_copy(e1_hbm.at[pl.ds(base, SCH)], idx1)
            descs = []
            for k in range(KSUB):
                sl = pl.ds(k * SUB, SUB)
                descs.append(pltpu.async_copy(p_hbm.at[idx0.at[sl]], bufa.at[sl], sem))
                descs.append(pltpu.async_copy(q_hbm.at[idx1.at[sl]], bufb.at[sl], sem))
            for d in descs:
                d.wait()
            pltpu.sync_copy(bufa, ga_hbm.at[pl.ds(base, SCH)])
            pltpu.sync_copy(bufb, gb_hbm.at[pl.ds(base, SCH)])
            return 0

        lax.fori_loop(0, n_super, body, 0)

    return gather_kernel


def _make_coords(E, N):
    T = E // NW
    n_super = T // SCH
    nv = SCH // 16
    mesh = plsc.VectorSubcoreMesh(core_axis_name="c", subcore_axis_name="s")

    @functools.partial(
        pl.kernel,
        mesh=mesh,
        out_type=jax.ShapeDtypeStruct((E * 8,), F32),
        compiler_params=pltpu.CompilerParams(needs_layout_passes=False),
        scratch_types=[
            pltpu.VMEM((N,), F32),
            pltpu.VMEM((N,), F32),
            pltpu.VMEM((N,), F32),
            pltpu.VMEM((SCH,), I32),
            pltpu.VMEM((SCH,), I32),
            pltpu.VMEM((SCH * 8,), F32),
            pltpu.SemaphoreType.DMA,
        ],
    )
    def coords_kernel(cx_hbm, cy_hbm, cz_hbm, e0_hbm, e1_hbm, cdn_hbm,
                      cxv, cyv, czv, idx0, idx1, stage, sem):
        wid = lax.axis_index("s") * NC + lax.axis_index("c")
        pltpu.sync_copy(cx_hbm, cxv)
        pltpu.sync_copy(cy_hbm, cyv)
        pltpu.sync_copy(cz_hbm, czv)
        zero16 = jnp.zeros((16,), F32)
        for u in range(SCH * 8 // 16):
            stage[pl.ds(u * 16, 16)] = zero16

        def body(j, _):
            base = wid * T + j * SCH
            pltpu.sync_copy(e0_hbm.at[pl.ds(base, SCH)], idx0)
            pltpu.sync_copy(e1_hbm.at[pl.ds(base, SCH)], idx1)
            for v in range(nv):
                i0 = idx0[pl.ds(v * 16, 16)]
                i1 = idx1[pl.ds(v * 16, 16)]
                dx = plsc.load_gather(cxv, [i0]) - plsc.load_gather(cxv, [i1])
                dy = plsc.load_gather(cyv, [i0]) - plsc.load_gather(cyv, [i1])
                dz = plsc.load_gather(czv, [i0]) - plsc.load_gather(czv, [i1])
                n2 = dx * dx + dy * dy + dz * dz
                rowb = (v * 16 + _iota16()) * 8
                plsc.store_scatter(stage, [rowb], dx)
                plsc.store_scatter(stage, [rowb + 1], dy)
                plsc.store_scatter(stage, [rowb + 2], dz)
                plsc.store_scatter(stage, [rowb + 3], n2)
            pltpu.sync_copy(stage, cdn_hbm.at[pl.ds(base * 8, SCH * 8)])
            return 0

        lax.fori_loop(0, n_super, body, 0)

    return coords_kernel


def _edge_mlp_body(ga_ref, gb_ref, cd_ref,
                   w1c_ref, b1_ref, w2_ref, b2_ref,
                   wc1_ref, bc1_ref, wc2_ref,
                   m_ref, tr_ref):
    cd = cd_ref[...]
    n2 = cd[:, 3:4]
    pre1 = ga_ref[...] + gb_ref[...] + n2 * w1c_ref[...] + b1_ref[...]
    x1 = jax.nn.silu(pre1)
    m = jax.nn.silu(jnp.dot(x1, w2_ref[...], preferred_element_type=F32)
                    + b2_ref[...])
    y = jax.nn.silu(jnp.dot(m, wc1_ref[...], preferred_element_type=F32)
                    + bc1_ref[...])
    s = jnp.dot(y, wc2_ref[...], preferred_element_type=F32)
    lane = lax.broadcasted_iota(I32, cd.shape, 1)
    tr_ref[...] = jnp.where(lane == 3, 1.0, cd * s)
    m_ref[...] = m


def _edge_mlp(ga, gb, cd, w1c, b1, W2, b2, Wc1, bc1, Wc2, blk):
    E, M = ga.shape
    full = lambda i: (0, 0)
    return pl.pallas_call(
        _edge_mlp_body,
        grid=(E // blk,),
        in_specs=[
            pl.BlockSpec((blk, M), lambda i: (i, 0)),
            pl.BlockSpec((blk, M), lambda i: (i, 0)),
            pl.BlockSpec((blk, 8), lambda i: (i, 0)),
            pl.BlockSpec((1, M), full),
            pl.BlockSpec((1, M), full),
            pl.BlockSpec((M, M), full),
            pl.BlockSpec((1, M), full),
            pl.BlockSpec((M, M), full),
            pl.BlockSpec((1, M), full),
            pl.BlockSpec((M, 1), full),
        ],
        out_specs=[
            pl.BlockSpec((blk, M), lambda i: (i, 0)),
            pl.BlockSpec((blk, 8), lambda i: (i, 0)),
        ],
        out_shape=[
            jax.ShapeDtypeStruct((E, M), F32),
            jax.ShapeDtypeStruct((E, 8), F32),
        ],
    )(ga, gb, cd, w1c.reshape(1, M), b1.reshape(1, M), W2,
      b2.reshape(1, M), Wc1, bc1.reshape(1, M), Wc2)


def _make_scatter(E, NP, M):
    T = E // NW
    n_super = T // SCH
    n_chunk = T // SUB
    idx_rows_pt = T // SUB
    rows_pt = NP // NS
    mesh = plsc.VectorSubcoreMesh(core_axis_name="c", subcore_axis_name="s")

    @functools.partial(
        pl.kernel,
        mesh=mesh,
        out_type=[
            jax.ShapeDtypeStruct((NC, NP, M), F32),
            jax.ShapeDtypeStruct((NC, NP, 128), F32),
        ],
        compiler_params=pltpu.CompilerParams(needs_layout_passes=False),
        scratch_types=[
            pltpu.VMEM((SUB,), I32),
            pltpu.VMEM((SUB, M), F32),
            pltpu.VMEM((SUB * 8,), F32),
            pltpu.VMEM((SUB, 128), F32),
            pltpu.VMEM_SHARED((NP, M), F32),
            pltpu.SemaphoreType.DMA,
        ],
    )
    def scatter_kernel(m_hbm, trf_hbm, e0_hbm, zm_hbm,
                       maggp_hbm, caggp_hbm,
                       idxc, mbuf, tbuf, stg0, sh, sem):
        cid = lax.axis_index("c")
        sid = lax.axis_index("s")
        wid = sid * NC + cid
        r0 = sid * rows_pt

        pltpu.sync_copy(zm_hbm.at[pl.ds(0, SUB)], stg0)
        pltpu.sync_copy(zm_hbm.at[pl.ds(r0, rows_pt)],
                        sh.at[pl.ds(r0, rows_pt)])
        plsc.subcore_barrier()

        def body_m(j, _):
            base = wid * T + j * SUB
            pltpu.sync_copy(e0_hbm.at[pl.ds(base, SUB)], idxc)
            pltpu.sync_copy(m_hbm.at[pl.ds(base, SUB)], mbuf)
            pltpu.sync_copy(mbuf, sh.at[idxc], add=True)
            return 0

        lax.fori_loop(0, n_chunk, body_m, 0)
        plsc.subcore_barrier()
        pltpu.sync_copy(sh.at[pl.ds(r0, rows_pt)],
                        maggp_hbm.at[cid, pl.ds(r0, rows_pt)])
        pltpu.sync_copy(zm_hbm.at[pl.ds(r0, rows_pt)],
                        sh.at[pl.ds(r0, rows_pt)])
        plsc.subcore_barrier()

        iota = _iota16()
        rloc = iota >> 3
        cloc = iota & 7

        def body_tr(j, _):
            base = wid * T + j * SUB
            pltpu.sync_copy(e0_hbm.at[pl.ds(base, SUB)], idxc)
            pltpu.sync_copy(trf_hbm.at[pl.ds(base * 8, SUB * 8)], tbuf)
            for u in range(SUB // 2):
                vals = tbuf[pl.ds(u * 16, 16)]
                plsc.store_scatter(stg0, [2 * u + rloc, cloc], vals)
            pltpu.sync_copy(stg0, sh.at[idxc], add=True)
            return 0

        lax.fori_loop(0, n_chunk, body_tr, 0)
        plsc.subcore_barrier()
        pltpu.sync_copy(sh.at[pl.ds(r0, rows_pt)],
                        caggp_hbm.at[cid, pl.ds(r0, rows_pt)])

    return scatter_kernel


def _node_mlp_body(cp_ref, h_ref, maggp_ref, caggp_ref,
                   wh1a_ref, wh1b_ref, bh1_ref, wh2_ref, bh2_ref,
                   co_ref, ho_ref):
    magg = maggp_ref[0] + maggp_ref[1]
    cagg = caggp_ref[0] + caggp_ref[1]
    counts = jnp.clip(cagg[:, 3:4], 1.0, None)
    co_ref[...] = cp_ref[...] + cagg[:, :8] / counts
    h = jax.nn.silu(jnp.dot(h_ref[...], wh1a_ref[...], preferred_element_type=F32)
                    + jnp.dot(magg, wh1b_ref[...], preferred_element_type=F32)
                    + bh1_ref[...])
    ho_ref[...] = jnp.dot(h, wh2_ref[...], preferred_element_type=F32) + bh2_ref[...]


def _node_mlp(coords_pad, hidden, maggp, caggp, Wh1a, Wh1b, bh1, Wh2, bh2, blk):
    n, hdim = hidden.shape
    m = Wh1a.shape[1]
    NP = maggp.shape[1]
    full = lambda i: (0, 0)
    return pl.pallas_call(
        _node_mlp_body,
        grid=(n // blk,),
        in_specs=[
            pl.BlockSpec((blk, 8), lambda i: (i, 0)),
            pl.BlockSpec((blk, hdim), lambda i: (i, 0)),
            pl.BlockSpec((NC, blk, m), lambda i: (0, i, 0)),
            pl.BlockSpec((NC, blk, 128), lambda i: (0, i, 0)),
            pl.BlockSpec((hdim, m), full),
            pl.BlockSpec((m, m), full),
            pl.BlockSpec((1, m), full),
            pl.BlockSpec((m, hdim), full),
            pl.BlockSpec((1, hdim), full),
        ],
        out_specs=[
            pl.BlockSpec((blk, 8), lambda i: (i, 0)),
            pl.BlockSpec((blk, hdim), lambda i: (i, 0)),
        ],
        out_shape=[
            jax.ShapeDtypeStruct((n, 8), F32),
            jax.ShapeDtypeStruct((n, hdim), F32),
        ],
    )(coords_pad, hidden, maggp, caggp, Wh1a, Wh1b,
      bh1.reshape(1, m), Wh2, bh2.reshape(1, hdim))


def kernel(coords, hidden, edges, W1, b1, W2, b2, Wc1, bc1, Wc2,
           Wh1, bh1, Wh2, bh2):
    N, H = hidden.shape
    E = edges.shape[1]
    M = W2.shape[0]

    e0 = edges[0]
    e1 = edges[1]
    e0r = e0.reshape(E // SUB, SUB)
    coords_pad = jnp.pad(coords, ((0, 0), (0, 5)))
    cx = coords[:, 0]
    cy = coords[:, 1]
    cz = coords[:, 2]

    W1a = W1[:H]
    W1b = W1[H:2 * H]
    w1c = W1[2 * H]
    Wh1a = Wh1[:H]
    Wh1b = Wh1[H:]

    P, Q = _precompute(hidden, W1a, W1b, blk=2000)
    ga, gb = _make_gather(E, M)(P, Q, e0, e1)
    cdn = _make_coords(E, N)(cx, cy, cz, e0, e1)
    cd = cdn.reshape(E, 8)

    m, tr = _edge_mlp(ga, gb, cd, w1c, b1, W2, b2, Wc1, bc1, Wc2, blk=2000)

    NP = ((N + NS * 8 - 1) // (NS * 8)) * NS * 8
    zm = jnp.zeros((NP, M), F32)
    maggp, caggp = _make_scatter(E, NP, M)(m, tr.reshape(E * 8), e0, zm)

    co8, hidden_out = _node_mlp(coords_pad, hidden, maggp, caggp,
                                Wh1a, Wh1b, bh1, Wh2, bh2, blk=2000)
    coords_out = co8[:, :3]
    return (coords_out, hidden_out)


# speedup vs baseline: 5.3249x; 1.2233x over previous
import functools

import jax
import jax.numpy as jnp
from jax import lax
from jax.experimental import pallas as pl
from jax.experimental.pallas import tpu as pltpu
from jax.experimental.pallas import tpu_sc as plsc

F32 = jnp.float32
I32 = jnp.int32

NC = 2
NS = 16
NW = NC * NS

SUB = 80
KSUB = 5
SCH = SUB * KSUB


def _iota16():
    return lax.iota(I32, 16)


def _precompute_body(h_ref, wa_ref, wb_ref, p_ref, q_ref):
    h = h_ref[...]
    p_ref[...] = jnp.dot(h, wa_ref[...], preferred_element_type=F32)
    q_ref[...] = jnp.dot(h, wb_ref[...], preferred_element_type=F32)


def _precompute(hidden, W1a, W1b, blk):
    n, hdim = hidden.shape
    m = W1a.shape[1]
    return pl.pallas_call(
        _precompute_body,
        grid=(n // blk,),
        in_specs=[
            pl.BlockSpec((blk, hdim), lambda i: (i, 0)),
            pl.BlockSpec((hdim, m), lambda i: (0, 0)),
            pl.BlockSpec((hdim, m), lambda i: (0, 0)),
        ],
        out_specs=[
            pl.BlockSpec((blk, m), lambda i: (i, 0)),
            pl.BlockSpec((blk, m), lambda i: (i, 0)),
        ],
        out_shape=[
            jax.ShapeDtypeStruct((n, m), F32),
            jax.ShapeDtypeStruct((n, m), F32),
        ],
    )(hidden, W1a, W1b)


def _make_gather(E, M):
    T = E // NW
    n_super = T // SCH
    mesh = plsc.VectorSubcoreMesh(core_axis_name="c", subcore_axis_name="s")

    @functools.partial(
        pl.kernel,
        mesh=mesh,
        out_type=[
            jax.ShapeDtypeStruct((E, M), F32),
            jax.ShapeDtypeStruct((E, M), F32),
        ],
        scratch_types=[
            pltpu.VMEM((SCH,), I32),
            pltpu.VMEM((SCH,), I32),
            pltpu.VMEM((SCH, M), F32),
            pltpu.VMEM((SCH, M), F32),
            pltpu.SemaphoreType.DMA,
        ],
    )
    def gather_kernel(p_hbm, q_hbm, e0_hbm, e1_hbm, ga_hbm, gb_hbm,
                      idx0, idx1, bufa, bufb, sem):
        wid = lax.axis_index("s") * NC + lax.axis_index("c")

        def body(j, _):
            base = wid * T + j * SCH
            pltpu.sync_copy(e0_hbm.at[pl.ds(base, SCH)], idx0)
            pltpu.sync_copy(e1_hbm.at[pl.ds(base, SCH)], idx1)
            descs = []
            for k in range(KSUB):
                sl = pl.ds(k * SUB, SUB)
                descs.append(pltpu.async_copy(p_hbm.at[idx0.at[sl]], bufa.at[sl], sem))
                descs.append(pltpu.async_copy(q_hbm.at[idx1.at[sl]], bufb.at[sl], sem))
            for d in descs:
                d.wait()
            pltpu.sync_copy(bufa, ga_hbm.at[pl.ds(base, SCH)])
            pltpu.sync_copy(bufb, gb_hbm.at[pl.ds(base, SCH)])
            return 0

        lax.fori_loop(0, n_super, body, 0)

    return gather_kernel


def _make_coords(E, N):
    T = E // NW
    n_super = T // SCH
    nv = SCH // 16
    mesh = plsc.VectorSubcoreMesh(core_axis_name="c", subcore_axis_name="s")

    @functools.partial(
        pl.kernel,
        mesh=mesh,
        out_type=jax.ShapeDtypeStruct((E * 8,), F32),
        compiler_params=pltpu.CompilerParams(needs_layout_passes=False),
        scratch_types=[
            pltpu.VMEM((N,), F32),
            pltpu.VMEM((N,), F32),
            pltpu.VMEM((N,), F32),
            pltpu.VMEM((SCH,), I32),
            pltpu.VMEM((SCH,), I32),
            pltpu.VMEM((SCH * 8,), F32),
            pltpu.SemaphoreType.DMA,
        ],
    )
    def coords_kernel(cx_hbm, cy_hbm, cz_hbm, e0_hbm, e1_hbm, cdn_hbm,
                      cxv, cyv, czv, idx0, idx1, stage, sem):
        wid = lax.axis_index("s") * NC + lax.axis_index("c")
        pltpu.sync_copy(cx_hbm, cxv)
        pltpu.sync_copy(cy_hbm, cyv)
        pltpu.sync_copy(cz_hbm, czv)
        zero16 = jnp.zeros((16,), F32)
        for u in range(SCH * 8 // 16):
            stage[pl.ds(u * 16, 16)] = zero16

        def body(j, _):
            base = wid * T + j * SCH
            pltpu.sync_copy(e0_hbm.at[pl.ds(base, SCH)], idx0)
            pltpu.sync_copy(e1_hbm.at[pl.ds(base, SCH)], idx1)
            for v in range(nv):
                i0 = idx0[pl.ds(v * 16, 16)]
                i1 = idx1[pl.ds(v * 16, 16)]
                dx = plsc.load_gather(cxv, [i0]) - plsc.load_gather(cxv, [i1])
                dy = plsc.load_gather(cyv, [i0]) - plsc.load_gather(cyv, [i1])
                dz = plsc.load_gather(czv, [i0]) - plsc.load_gather(czv, [i1])
                n2 = dx * dx + dy * dy + dz * dz
                rowb = (v * 16 + _iota16()) * 8
                plsc.store_scatter(stage, [rowb], dx)
                plsc.store_scatter(stage, [rowb + 1], dy)
                plsc.store_scatter(stage, [rowb + 2], dz)
                plsc.store_scatter(stage, [rowb + 3], n2)
            pltpu.sync_copy(stage, cdn_hbm.at[pl.ds(base * 8, SCH * 8)])
            return 0

        lax.fori_loop(0, n_super, body, 0)

    return coords_kernel


def _edge_mlp_body(ga_ref, gb_ref, cd_ref,
                   w1c_ref, b1_ref, w2_ref, b2_ref,
                   wc1_ref, bc1_ref, wc2_ref,
                   m_ref, tr_ref):
    cd = cd_ref[...]
    n2 = cd[:, 3:4]
    pre1 = ga_ref[...] + gb_ref[...] + n2 * w1c_ref[...] + b1_ref[...]
    x1 = jax.nn.silu(pre1)
    m = jax.nn.silu(jnp.dot(x1, w2_ref[...], preferred_element_type=F32)
                    + b2_ref[...])
    y = jax.nn.silu(jnp.dot(m, wc1_ref[...], preferred_element_type=F32)
                    + bc1_ref[...])
    s = jnp.dot(y, wc2_ref[...], preferred_element_type=F32)
    lane = lax.broadcasted_iota(I32, cd.shape, 1)
    tr_ref[...] = jnp.where(lane == 3, 1.0, cd * s)
    m_ref[...] = m


def _edge_mlp(ga, gb, cd, w1c, b1, W2, b2, Wc1, bc1, Wc2, blk):
    E, M = ga.shape
    full = lambda i: (0, 0)
    return pl.pallas_call(
        _edge_mlp_body,
        grid=(E // blk,),
        in_specs=[
            pl.BlockSpec((blk, M), lambda i: (i, 0)),
            pl.BlockSpec((blk, M), lambda i: (i, 0)),
            pl.BlockSpec((blk, 8), lambda i: (i, 0)),
            pl.BlockSpec((1, M), full),
            pl.BlockSpec((1, M), full),
            pl.BlockSpec((M, M), full),
            pl.BlockSpec((1, M), full),
            pl.BlockSpec((M, M), full),
            pl.BlockSpec((1, M), full),
            pl.BlockSpec((M, 1), full),
        ],
        out_specs=[
            pl.BlockSpec((blk, M), lambda i: (i, 0)),
            pl.BlockSpec((blk, 8), lambda i: (i, 0)),
        ],
        out_shape=[
            jax.ShapeDtypeStruct((E, M), F32),
            jax.ShapeDtypeStruct((E, 8), F32),
        ],
    )(ga, gb, cd, w1c.reshape(1, M), b1.reshape(1, M), W2,
      b2.reshape(1, M), Wc1, bc1.reshape(1, M), Wc2)


def _make_scatter(E, NP, M):
    T = E // NW
    n_chunk = T // SUB
    rows_pt = NP // NS
    mesh = plsc.VectorSubcoreMesh(core_axis_name="c", subcore_axis_name="s")

    @functools.partial(
        pl.kernel,
        mesh=mesh,
        out_type=[
            jax.ShapeDtypeStruct((NC, NP, M), F32),
            jax.ShapeDtypeStruct((NC, NP, 128), F32),
        ],
        compiler_params=pltpu.CompilerParams(needs_layout_passes=False),
        scratch_types=[
            pltpu.VMEM((SUB,), I32),
            pltpu.VMEM((SUB,), I32),
            pltpu.VMEM((SUB, M), F32),
            pltpu.VMEM((SUB, M), F32),
            pltpu.VMEM((SUB * 8,), F32),
            pltpu.VMEM((SUB * 8,), F32),
            pltpu.VMEM((SUB, 128), F32),
            pltpu.VMEM((SUB, 128), F32),
            pltpu.VMEM_SHARED((NP, M), F32),
            pltpu.SemaphoreType.DMA,
            pltpu.SemaphoreType.DMA,
            pltpu.SemaphoreType.DMA,
            pltpu.SemaphoreType.DMA,
        ],
    )
    def scatter_kernel(m_hbm, trf_hbm, e0_hbm, zm_hbm,
                       maggp_hbm, caggp_hbm,
                       idxc0, idxc1, mbuf0, mbuf1, tbuf0, tbuf1, stg0, stg1,
                       sh, semL0, semL1, semS0, semS1):
        cid = lax.axis_index("c")
        sid = lax.axis_index("s")
        wid = sid * NC + cid
        r0 = sid * rows_pt
        idxc = (idxc0, idxc1)
        mbuf = (mbuf0, mbuf1)
        tbuf = (tbuf0, tbuf1)
        stg = (stg0, stg1)
        semL = (semL0, semL1)
        semS = (semS0, semS1)

        pltpu.sync_copy(zm_hbm.at[pl.ds(0, SUB)], stg0)
        pltpu.sync_copy(zm_hbm.at[pl.ds(0, SUB)], stg1)
        pltpu.sync_copy(zm_hbm.at[pl.ds(r0, rows_pt)],
                        sh.at[pl.ds(r0, rows_pt)])
        plsc.subcore_barrier()

        def load1(j, b):
            base = wid * T + j * SUB
            pltpu.async_copy(e0_hbm.at[pl.ds(base, SUB)], idxc[b], semL[b])
            pltpu.async_copy(m_hbm.at[pl.ds(base, SUB)], mbuf[b], semL[b])

        def drain_load1(b):
            pltpu.make_async_copy(e0_hbm.at[pl.ds(0, SUB)], idxc[b], semL[b]).wait()
            pltpu.make_async_copy(m_hbm.at[pl.ds(0, SUB)], mbuf[b], semL[b]).wait()

        def fire_stream1(b):
            pltpu.async_copy(mbuf[b], sh.at[idxc[b]], semS[b], add=True)

        def drain_stream1(b):
            pltpu.make_async_copy(mbuf[b], sh.at[idxc[b]], semS[b]).wait()

        def half1(t, j, b, guard):
            if guard:
                drain_stream1(1 - b)
            else:
                @pl.when(t > 0)
                def _():
                    drain_stream1(1 - b)
            load1(j + 1, 1 - b)
            drain_load1(b)
            fire_stream1(b)

        load1(0, 0)

        def body1(t, _):
            half1(t, 2 * t, 0, False)
            half1(t, 2 * t + 1, 1, True)
            return 0

        lax.fori_loop(0, (n_chunk - 1) // 2, body1, 0)
        drain_stream1(1)
        drain_load1(0)
        fire_stream1(0)
        drain_stream1(0)

        plsc.subcore_barrier()
        pltpu.sync_copy(sh.at[pl.ds(r0, rows_pt)],
                        maggp_hbm.at[cid, pl.ds(r0, rows_pt)])
        pltpu.sync_copy(zm_hbm.at[pl.ds(r0, rows_pt)],
                        sh.at[pl.ds(r0, rows_pt)])
        plsc.subcore_barrier()

        iota = _iota16()
        rloc = iota >> 3
        cloc = iota & 7

        def load2(j, b):
            base = wid * T + j * SUB
            pltpu.async_copy(e0_hbm.at[pl.ds(base, SUB)], idxc[b], semL[b])
            pltpu.async_copy(trf_hbm.at[pl.ds(base * 8, SUB * 8)], tbuf[b], semL[b])

        def drain_load2(b):
            pltpu.make_async_copy(e0_hbm.at[pl.ds(0, SUB)], idxc[b], semL[b]).wait()
            pltpu.make_async_copy(trf_hbm.at[pl.ds(0, SUB * 8)], tbuf[b], semL[b]).wait()

        def fill(b):
            for u in range(SUB // 2):
                vals = tbuf[b][pl.ds(u * 16, 16)]
                plsc.store_scatter(stg[b], [2 * u + rloc, cloc], vals)

        def fire_stream2(b):
            pltpu.async_copy(stg[b], sh.at[idxc[b]], semS[b], add=True)

        def drain_stream2(b):
            pltpu.make_async_copy(stg[b], sh.at[idxc[b]], semS[b]).wait()

        def half2(t, j, b, guard):
            drain_load2(b)
            fill(b)
            fire_stream2(b)
            if guard:
                drain_stream2(1 - b)
            else:
                @pl.when(t > 0)
                def _():
                    drain_stream2(1 - b)
            load2(j + 1, 1 - b)

        load2(0, 0)

        def body2(t, _):
            half2(t, 2 * t, 0, False)
            half2(t, 2 * t + 1, 1, True)
            return 0

        lax.fori_loop(0, (n_chunk - 1) // 2, body2, 0)
        drain_load2(0)
        fill(0)
        fire_stream2(0)
        drain_stream2(1)
        drain_stream2(0)

        plsc.subcore_barrier()
        pltpu.sync_copy(sh.at[pl.ds(r0, rows_pt)],
                        caggp_hbm.at[cid, pl.ds(r0, rows_pt)])

    return scatter_kernel


def _node_mlp_body(cp_ref, h_ref, maggp_ref, caggp_ref,
                   wh1a_ref, wh1b_ref, bh1_ref, wh2_ref, bh2_ref,
                   co_ref, ho_ref):
    magg = maggp_ref[0] + maggp_ref[1]
    cagg = caggp_ref[0] + caggp_ref[1]
    counts = jnp.clip(cagg[:, 3:4], 1.0, None)
    co_ref[...] = cp_ref[...] + cagg[:, :8] / counts
    h = jax.nn.silu(jnp.dot(h_ref[...], wh1a_ref[...], preferred_element_type=F32)
                    + jnp.dot(magg, wh1b_ref[...], preferred_element_type=F32)
                    + bh1_ref[...])
    ho_ref[...] = jnp.dot(h, wh2_ref[...], preferred_element_type=F32) + bh2_ref[...]


def _node_mlp(coords_pad, hidden, maggp, caggp, Wh1a, Wh1b, bh1, Wh2, bh2, blk):
    n, hdim = hidden.shape
    m = Wh1a.shape[1]
    NP = maggp.shape[1]
    full = lambda i: (0, 0)
    return pl.pallas_call(
        _node_mlp_body,
        grid=(n // blk,),
        in_specs=[
            pl.BlockSpec((blk, 8), lambda i: (i, 0)),
            pl.BlockSpec((blk, hdim), lambda i: (i, 0)),
            pl.BlockSpec((NC, blk, m), lambda i: (0, i, 0)),
            pl.BlockSpec((NC, blk, 128), lambda i: (0, i, 0)),
            pl.BlockSpec((hdim, m), full),
            pl.BlockSpec((m, m), full),
            pl.BlockSpec((1, m), full),
            pl.BlockSpec((m, hdim), full),
            pl.BlockSpec((1, hdim), full),
        ],
        out_specs=[
            pl.BlockSpec((blk, 8), lambda i: (i, 0)),
            pl.BlockSpec((blk, hdim), lambda i: (i, 0)),
        ],
        out_shape=[
            jax.ShapeDtypeStruct((n, 8), F32),
            jax.ShapeDtypeStruct((n, hdim), F32),
        ],
    )(coords_pad, hidden, maggp, caggp, Wh1a, Wh1b,
      bh1.reshape(1, m), Wh2, bh2.reshape(1, hdim))


def kernel(coords, hidden, edges, W1, b1, W2, b2, Wc1, bc1, Wc2,
           Wh1, bh1, Wh2, bh2):
    N, H = hidden.shape
    E = edges.shape[1]
    M = W2.shape[0]

    e0 = edges[0]
    e1 = edges[1]
    e0r = e0.reshape(E // SUB, SUB)
    coords_pad = jnp.pad(coords, ((0, 0), (0, 5)))
    cx = coords[:, 0]
    cy = coords[:, 1]
    cz = coords[:, 2]

    W1a = W1[:H]
    W1b = W1[H:2 * H]
    w1c = W1[2 * H]
    Wh1a = Wh1[:H]
    Wh1b = Wh1[H:]

    P, Q = _precompute(hidden, W1a, W1b, blk=2000)
    ga, gb = _make_gather(E, M)(P, Q, e0, e1)
    cdn = _make_coords(E, N)(cx, cy, cz, e0, e1)
    cd = cdn.reshape(E, 8)

    m, tr = _edge_mlp(ga, gb, cd, w1c, b1, W2, b2, Wc1, bc1, Wc2, blk=2000)

    NP = ((N + NS * 8 - 1) // (NS * 8)) * NS * 8
    zm = jnp.zeros((NP, M), F32)
    maggp, caggp = _make_scatter(E, NP, M)(m, tr.reshape(E * 8), e0, zm)

    co8, hidden_out = _node_mlp(coords_pad, hidden, maggp, caggp,
                                Wh1a, Wh1b, bh1, Wh2, bh2, blk=2000)
    coords_out = co8[:, :3]
    return (coords_out, hidden_out)


# speedup vs baseline: 5.4634x; 1.0260x over previous
import functools

import jax
import jax.numpy as jnp
from jax import lax
from jax.experimental import pallas as pl
from jax.experimental.pallas import tpu as pltpu
from jax.experimental.pallas import tpu_sc as plsc

F32 = jnp.float32
I32 = jnp.int32

NC = 2
NS = 16
NW = NC * NS

SUB = 80
KSUB = 5
SCH = SUB * KSUB


def _iota16():
    return lax.iota(I32, 16)


def _precompute_body(h_ref, wa_ref, wb_ref, p_ref, q_ref):
    h = h_ref[...]
    p_ref[...] = jnp.dot(h, wa_ref[...], preferred_element_type=F32)
    q_ref[...] = jnp.dot(h, wb_ref[...], preferred_element_type=F32)


def _precompute(hidden, W1a, W1b, blk):
    n, hdim = hidden.shape
    m = W1a.shape[1]
    return pl.pallas_call(
        _precompute_body,
        grid=(n // blk,),
        in_specs=[
            pl.BlockSpec((blk, hdim), lambda i: (i, 0)),
            pl.BlockSpec((hdim, m), lambda i: (0, 0)),
            pl.BlockSpec((hdim, m), lambda i: (0, 0)),
        ],
        out_specs=[
            pl.BlockSpec((blk, m), lambda i: (i, 0)),
            pl.BlockSpec((blk, m), lambda i: (i, 0)),
        ],
        out_shape=[
            jax.ShapeDtypeStruct((n, m), F32),
            jax.ShapeDtypeStruct((n, m), F32),
        ],
    )(hidden, W1a, W1b)


GSUB = 40
GSCH = 200


def _make_gather(E, M):
    T = E // NW
    n_super = T // GSCH
    mesh = plsc.VectorSubcoreMesh(core_axis_name="c", subcore_axis_name="s")

    @functools.partial(
        pl.kernel,
        mesh=mesh,
        out_type=[
            jax.ShapeDtypeStruct((E, M), F32),
            jax.ShapeDtypeStruct((E, M), F32),
        ],
        scratch_types=[
            pltpu.VMEM((GSCH,), I32),
            pltpu.VMEM((GSCH,), I32),
            pltpu.VMEM((GSCH,), I32),
            pltpu.VMEM((GSCH,), I32),
            pltpu.VMEM((GSCH, M), F32),
            pltpu.VMEM((GSCH, M), F32),
            pltpu.VMEM((GSCH, M), F32),
            pltpu.VMEM((GSCH, M), F32),
            pltpu.SemaphoreType.DMA,
            pltpu.SemaphoreType.DMA,
            pltpu.SemaphoreType.DMA,
            pltpu.SemaphoreType.DMA,
            pltpu.SemaphoreType.DMA,
            pltpu.SemaphoreType.DMA,
        ],
    )
    def gather_kernel(p_hbm, q_hbm, e0_hbm, e1_hbm, ga_hbm, gb_hbm,
                      idx0a, idx0b, idx1a, idx1b, bufa0, bufa1, bufb0, bufb1,
                      semI0, semI1, semG0, semG1, semO0, semO1):
        wid = lax.axis_index("s") * NC + lax.axis_index("c")
        idx0 = (idx0a, idx0b)
        idx1 = (idx1a, idx1b)
        bufa = (bufa0, bufa1)
        bufb = (bufb0, bufb1)
        semI = (semI0, semI1)
        semG = (semG0, semG1)
        semO = (semO0, semO1)

        def fire_idx(j, b):
            base = wid * T + j * GSCH
            pltpu.async_copy(e0_hbm.at[pl.ds(base, GSCH)], idx0[b], semI[b])
            pltpu.async_copy(e1_hbm.at[pl.ds(base, GSCH)], idx1[b], semI[b])

        def drain_idx(b):
            pltpu.make_async_copy(e0_hbm.at[pl.ds(0, GSCH)], idx0[b], semI[b]).wait()
            pltpu.make_async_copy(e1_hbm.at[pl.ds(0, GSCH)], idx1[b], semI[b]).wait()

        def fire_gathers(b):
            for k in range(GSCH // GSUB):
                sl = pl.ds(k * GSUB, GSUB)
                pltpu.async_copy(p_hbm.at[idx0[b].at[sl]], bufa[b].at[sl], semG[b])
                pltpu.async_copy(q_hbm.at[idx1[b].at[sl]], bufb[b].at[sl], semG[b])

        def drain_gathers(b):
            for k in range(GSCH // GSUB):
                sl = pl.ds(k * GSUB, GSUB)
                pltpu.make_async_copy(p_hbm.at[idx0[b].at[sl]], bufa[b].at[sl], semG[b]).wait()
                pltpu.make_async_copy(q_hbm.at[idx1[b].at[sl]], bufb[b].at[sl], semG[b]).wait()

        def fire_out(j, b):
            base = wid * T + j * GSCH
            pltpu.async_copy(bufa[b], ga_hbm.at[pl.ds(base, GSCH)], semO[b])
            pltpu.async_copy(bufb[b], gb_hbm.at[pl.ds(base, GSCH)], semO[b])

        def drain_out(b):
            pltpu.make_async_copy(bufa[b], ga_hbm.at[pl.ds(0, GSCH)], semO[b]).wait()
            pltpu.make_async_copy(bufb[b], gb_hbm.at[pl.ds(0, GSCH)], semO[b]).wait()

        def half(t, j, b, first):
            drain_idx(b)

            @pl.when(t > 0)
            def _():
                drain_out(b)

            fire_gathers(b)
            if first:
                @pl.when(t > 0)
                def _():
                    drain_gathers(1 - b)
                    fire_out(j - 1, 1 - b)
            else:
                drain_gathers(1 - b)
                fire_out(j - 1, 1 - b)
            @pl.when(j + 1 < n_super)
            def _():
                fire_idx(j + 1, 1 - b)

        fire_idx(0, 0)

        def body(t, _):
            half(t, 2 * t, 0, True)
            half(t, 2 * t + 1, 1, False)
            return 0

        lax.fori_loop(0, n_super // 2, body, 0)
        drain_gathers(1)
        fire_out(n_super - 1, 1)
        drain_out(0)
        drain_out(1)

    return gather_kernel


def _make_coords(E, N):
    T = E // NW
    n_super = T // SCH
    nv = SCH // 16
    mesh = plsc.VectorSubcoreMesh(core_axis_name="c", subcore_axis_name="s")

    @functools.partial(
        pl.kernel,
        mesh=mesh,
        out_type=jax.ShapeDtypeStruct((E * 8,), F32),
        compiler_params=pltpu.CompilerParams(needs_layout_passes=False),
        scratch_types=[
            pltpu.VMEM((N,), F32),
            pltpu.VMEM((N,), F32),
            pltpu.VMEM((N,), F32),
            pltpu.VMEM((SCH,), I32),
            pltpu.VMEM((SCH,), I32),
            pltpu.VMEM((SCH * 8,), F32),
            pltpu.SemaphoreType.DMA,
        ],
    )
    def coords_kernel(cx_hbm, cy_hbm, cz_hbm, e0_hbm, e1_hbm, cdn_hbm,
                      cxv, cyv, czv, idx0, idx1, stage, sem):
        wid = lax.axis_index("s") * NC + lax.axis_index("c")
        pltpu.sync_copy(cx_hbm, cxv)
        pltpu.sync_copy(cy_hbm, cyv)
        pltpu.sync_copy(cz_hbm, czv)
        zero16 = jnp.zeros((16,), F32)
        for u in range(SCH * 8 // 16):
            stage[pl.ds(u * 16, 16)] = zero16

        def body(j, _):
            base = wid * T + j * SCH
            pltpu.sync_copy(e0_hbm.at[pl.ds(base, SCH)], idx0)
            pltpu.sync_copy(e1_hbm.at[pl.ds(base, SCH)], idx1)
            for v in range(nv):
                i0 = idx0[pl.ds(v * 16, 16)]
                i1 = idx1[pl.ds(v * 16, 16)]
                dx = plsc.load_gather(cxv, [i0]) - plsc.load_gather(cxv, [i1])
                dy = plsc.load_gather(cyv, [i0]) - plsc.load_gather(cyv, [i1])
                dz = plsc.load_gather(czv, [i0]) - plsc.load_gather(czv, [i1])
                n2 = dx * dx + dy * dy + dz * dz
                rowb = (v * 16 + _iota16()) * 8
                plsc.store_scatter(stage, [rowb], dx)
                plsc.store_scatter(stage, [rowb + 1], dy)
                plsc.store_scatter(stage, [rowb + 2], dz)
                plsc.store_scatter(stage, [rowb + 3], n2)
            pltpu.sync_copy(stage, cdn_hbm.at[pl.ds(base * 8, SCH * 8)])
            return 0

        lax.fori_loop(0, n_super, body, 0)

    return coords_kernel


def _edge_mlp_body(ga_ref, gb_ref, cd_ref,
                   w1c_ref, b1_ref, w2_ref, b2_ref,
                   wc1_ref, bc1_ref, wc2_ref,
                   m_ref, tr_ref):
    cd = cd_ref[...]
    n2 = cd[:, 3:4]
    pre1 = ga_ref[...] + gb_ref[...] + n2 * w1c_ref[...] + b1_ref[...]
    x1 = jax.nn.silu(pre1)
    m = jax.nn.silu(jnp.dot(x1, w2_ref[...], preferred_element_type=F32)
                    + b2_ref[...])
    y = jax.nn.silu(jnp.dot(m, wc1_ref[...], preferred_element_type=F32)
                    + bc1_ref[...])
    s = jnp.dot(y, wc2_ref[...], preferred_element_type=F32)
    lane = lax.broadcasted_iota(I32, cd.shape, 1)
    tr_ref[...] = jnp.where(lane == 3, 1.0, cd * s)
    m_ref[...] = m


def _edge_mlp(ga, gb, cd, w1c, b1, W2, b2, Wc1, bc1, Wc2, blk):
    E, M = ga.shape
    full = lambda i: (0, 0)
    return pl.pallas_call(
        _edge_mlp_body,
        grid=(E // blk,),
        in_specs=[
            pl.BlockSpec((blk, M), lambda i: (i, 0)),
            pl.BlockSpec((blk, M), lambda i: (i, 0)),
            pl.BlockSpec((blk, 8), lambda i: (i, 0)),
            pl.BlockSpec((1, M), full),
            pl.BlockSpec((1, M), full),
            pl.BlockSpec((M, M), full),
            pl.BlockSpec((1, M), full),
            pl.BlockSpec((M, M), full),
            pl.BlockSpec((1, M), full),
            pl.BlockSpec((M, 1), full),
        ],
        out_specs=[
            pl.BlockSpec((blk, M), lambda i: (i, 0)),
            pl.BlockSpec((blk, 8), lambda i: (i, 0)),
        ],
        out_shape=[
            jax.ShapeDtypeStruct((E, M), F32),
            jax.ShapeDtypeStruct((E, 8), F32),
        ],
    )(ga, gb, cd, w1c.reshape(1, M), b1.reshape(1, M), W2,
      b2.reshape(1, M), Wc1, bc1.reshape(1, M), Wc2)


def _make_scatter(E, NP, M):
    T = E // NW
    n_chunk = T // SUB
    rows_pt = NP // NS
    mesh = plsc.VectorSubcoreMesh(core_axis_name="c", subcore_axis_name="s")

    @functools.partial(
        pl.kernel,
        mesh=mesh,
        out_type=[
            jax.ShapeDtypeStruct((NC, NP, M), F32),
            jax.ShapeDtypeStruct((NC, NP, 128), F32),
        ],
        compiler_params=pltpu.CompilerParams(needs_layout_passes=False),
        scratch_types=[
            pltpu.VMEM((SUB,), I32),
            pltpu.VMEM((SUB,), I32),
            pltpu.VMEM((SUB, M), F32),
            pltpu.VMEM((SUB, M), F32),
            pltpu.VMEM((SUB * 8,), F32),
            pltpu.VMEM((SUB * 8,), F32),
            pltpu.VMEM((SUB, 128), F32),
            pltpu.VMEM((SUB, 128), F32),
            pltpu.VMEM_SHARED((NP, M), F32),
            pltpu.SemaphoreType.DMA,
            pltpu.SemaphoreType.DMA,
            pltpu.SemaphoreType.DMA,
            pltpu.SemaphoreType.DMA,
        ],
    )
    def scatter_kernel(m_hbm, trf_hbm, e0_hbm, zm_hbm,
                       maggp_hbm, caggp_hbm,
                       idxc0, idxc1, mbuf0, mbuf1, tbuf0, tbuf1, stg0, stg1,
                       sh, semL0, semL1, semS0, semS1):
        cid = lax.axis_index("c")
        sid = lax.axis_index("s")
        wid = sid * NC + cid
        r0 = sid * rows_pt
        idxc = (idxc0, idxc1)
        mbuf = (mbuf0, mbuf1)
        tbuf = (tbuf0, tbuf1)
        stg = (stg0, stg1)
        semL = (semL0, semL1)
        semS = (semS0, semS1)

        pltpu.sync_copy(zm_hbm.at[pl.ds(0, SUB)], stg0)
        pltpu.sync_copy(zm_hbm.at[pl.ds(0, SUB)], stg1)
        pltpu.sync_copy(zm_hbm.at[pl.ds(r0, rows_pt)],
                        sh.at[pl.ds(r0, rows_pt)])
        plsc.subcore_barrier()

        def load1(j, b):
            base = wid * T + j * SUB
            pltpu.async_copy(e0_hbm.at[pl.ds(base, SUB)], idxc[b], semL[b])
            pltpu.async_copy(m_hbm.at[pl.ds(base, SUB)], mbuf[b], semL[b])

        def drain_load1(b):
            pltpu.make_async_copy(e0_hbm.at[pl.ds(0, SUB)], idxc[b], semL[b]).wait()
            pltpu.make_async_copy(m_hbm.at[pl.ds(0, SUB)], mbuf[b], semL[b]).wait()

        def fire_stream1(b):
            pltpu.async_copy(mbuf[b], sh.at[idxc[b]], semS[b], add=True)

        def drain_stream1(b):
            pltpu.make_async_copy(mbuf[b], sh.at[idxc[b]], semS[b]).wait()

        def half1(t, j, b, guard):
            if guard:
                drain_stream1(1 - b)
            else:
                @pl.when(t > 0)
                def _():
                    drain_stream1(1 - b)
            load1(j + 1, 1 - b)
            drain_load1(b)
            fire_stream1(b)

        load1(0, 0)

        def body1(t, _):
            half1(t, 2 * t, 0, False)
            half1(t, 2 * t + 1, 1, True)
            return 0

        lax.fori_loop(0, (n_chunk - 1) // 2, body1, 0)
        drain_stream1(1)
        drain_load1(0)
        fire_stream1(0)
        drain_stream1(0)

        plsc.subcore_barrier()
        pltpu.sync_copy(sh.at[pl.ds(r0, rows_pt)],
                        maggp_hbm.at[cid, pl.ds(r0, rows_pt)])
        pltpu.sync_copy(zm_hbm.at[pl.ds(r0, rows_pt)],
                        sh.at[pl.ds(r0, rows_pt)])
        plsc.subcore_barrier()

        iota = _iota16()
        rloc = iota >> 3
        cloc = iota & 7

        def load2(j, b):
            base = wid * T + j * SUB
            pltpu.async_copy(e0_hbm.at[pl.ds(base, SUB)], idxc[b], semL[b])
            pltpu.async_copy(trf_hbm.at[pl.ds(base * 8, SUB * 8)], tbuf[b], semL[b])

        def drain_load2(b):
            pltpu.make_async_copy(e0_hbm.at[pl.ds(0, SUB)], idxc[b], semL[b]).wait()
            pltpu.make_async_copy(trf_hbm.at[pl.ds(0, SUB * 8)], tbuf[b], semL[b]).wait()

        def fill(b):
            for u in range(SUB // 2):
                vals = tbuf[b][pl.ds(u * 16, 16)]
                plsc.store_scatter(stg[b], [2 * u + rloc, cloc], vals)

        def fire_stream2(b):
            pltpu.async_copy(stg[b], sh.at[idxc[b]], semS[b], add=True)

        def drain_stream2(b):
            pltpu.make_async_copy(stg[b], sh.at[idxc[b]], semS[b]).wait()

        def half2(t, j, b, guard):
            drain_load2(b)
            fill(b)
            fire_stream2(b)
            if guard:
                drain_stream2(1 - b)
            else:
                @pl.when(t > 0)
                def _():
                    drain_stream2(1 - b)
            load2(j + 1, 1 - b)

        load2(0, 0)

        def body2(t, _):
            half2(t, 2 * t, 0, False)
            half2(t, 2 * t + 1, 1, True)
            return 0

        lax.fori_loop(0, (n_chunk - 1) // 2, body2, 0)
        drain_load2(0)
        fill(0)
        fire_stream2(0)
        drain_stream2(1)
        drain_stream2(0)

        plsc.subcore_barrier()
        pltpu.sync_copy(sh.at[pl.ds(r0, rows_pt)],
                        caggp_hbm.at[cid, pl.ds(r0, rows_pt)])

    return scatter_kernel


def _node_mlp_body(cp_ref, h_ref, maggp_ref, caggp_ref,
                   wh1a_ref, wh1b_ref, bh1_ref, wh2_ref, bh2_ref,
                   co_ref, ho_ref):
    magg = maggp_ref[0] + maggp_ref[1]
    cagg = caggp_ref[0] + caggp_ref[1]
    counts = jnp.clip(cagg[:, 3:4], 1.0, None)
    co_ref[...] = cp_ref[...] + cagg[:, :8] / counts
    h = jax.nn.silu(jnp.dot(h_ref[...], wh1a_ref[...], preferred_element_type=F32)
                    + jnp.dot(magg, wh1b_ref[...], preferred_element_type=F32)
                    + bh1_ref[...])
    ho_ref[...] = jnp.dot(h, wh2_ref[...], preferred_element_type=F32) + bh2_ref[...]


def _node_mlp(coords_pad, hidden, maggp, caggp, Wh1a, Wh1b, bh1, Wh2, bh2, blk):
    n, hdim = hidden.shape
    m = Wh1a.shape[1]
    NP = maggp.shape[1]
    full = lambda i: (0, 0)
    return pl.pallas_call(
        _node_mlp_body,
        grid=(n // blk,),
        in_specs=[
            pl.BlockSpec((blk, 8), lambda i: (i, 0)),
            pl.BlockSpec((blk, hdim), lambda i: (i, 0)),
            pl.BlockSpec((NC, blk, m), lambda i: (0, i, 0)),
            pl.BlockSpec((NC, blk, 128), lambda i: (0, i, 0)),
            pl.BlockSpec((hdim, m), full),
            pl.BlockSpec((m, m), full),
            pl.BlockSpec((1, m), full),
            pl.BlockSpec((m, hdim), full),
            pl.BlockSpec((1, hdim), full),
        ],
        out_specs=[
            pl.BlockSpec((blk, 8), lambda i: (i, 0)),
            pl.BlockSpec((blk, hdim), lambda i: (i, 0)),
        ],
        out_shape=[
            jax.ShapeDtypeStruct((n, 8), F32),
            jax.ShapeDtypeStruct((n, hdim), F32),
        ],
    )(coords_pad, hidden, maggp, caggp, Wh1a, Wh1b,
      bh1.reshape(1, m), Wh2, bh2.reshape(1, hdim))


def kernel(coords, hidden, edges, W1, b1, W2, b2, Wc1, bc1, Wc2,
           Wh1, bh1, Wh2, bh2):
    N, H = hidden.shape
    E = edges.shape[1]
    M = W2.shape[0]

    e0 = edges[0]
    e1 = edges[1]
    e0r = e0.reshape(E // SUB, SUB)
    coords_pad = jnp.pad(coords, ((0, 0), (0, 5)))
    cx = coords[:, 0]
    cy = coords[:, 1]
    cz = coords[:, 2]

    W1a = W1[:H]
    W1b = W1[H:2 * H]
    w1c = W1[2 * H]
    Wh1a = Wh1[:H]
    Wh1b = Wh1[H:]

    P, Q = _precompute(hidden, W1a, W1b, blk=2000)
    ga, gb = _make_gather(E, M)(P, Q, e0, e1)
    cdn = _make_coords(E, N)(cx, cy, cz, e0, e1)
    cd = cdn.reshape(E, 8)

    m, tr = _edge_mlp(ga, gb, cd, w1c, b1, W2, b2, Wc1, bc1, Wc2, blk=2000)

    NP = ((N + NS * 8 - 1) // (NS * 8)) * NS * 8
    zm = jnp.zeros((NP, M), F32)
    maggp, caggp = _make_scatter(E, NP, M)(m, tr.reshape(E * 8), e0, zm)

    co8, hidden_out = _node_mlp(coords_pad, hidden, maggp, caggp,
                                Wh1a, Wh1b, bh1, Wh2, bh2, blk=2000)
    coords_out = co8[:, :3]
    return (coords_out, hidden_out)


# speedup vs baseline: 5.6750x; 1.0387x over previous
import functools

import jax
import jax.numpy as jnp
from jax import lax
from jax.experimental import pallas as pl
from jax.experimental.pallas import tpu as pltpu
from jax.experimental.pallas import tpu_sc as plsc

F32 = jnp.float32
BF16 = jnp.bfloat16
I32 = jnp.int32

NC = 2
NS = 16
NW = NC * NS

SUB = 80
KSUB = 5
SCH = SUB * KSUB


def _iota16():
    return lax.iota(I32, 16)


def _precompute_body(h_ref, wa_ref, wb_ref, p_ref, q_ref):
    h = h_ref[...]
    p_ref[...] = jnp.dot(h, wa_ref[...], preferred_element_type=F32)
    q_ref[...] = jnp.dot(h, wb_ref[...], preferred_element_type=F32)


def _precompute(hidden, W1a, W1b, blk):
    n, hdim = hidden.shape
    m = W1a.shape[1]
    return pl.pallas_call(
        _precompute_body,
        grid=(n // blk,),
        in_specs=[
            pl.BlockSpec((blk, hdim), lambda i: (i, 0)),
            pl.BlockSpec((hdim, m), lambda i: (0, 0)),
            pl.BlockSpec((hdim, m), lambda i: (0, 0)),
        ],
        out_specs=[
            pl.BlockSpec((blk, m), lambda i: (i, 0)),
            pl.BlockSpec((blk, m), lambda i: (i, 0)),
        ],
        out_shape=[
            jax.ShapeDtypeStruct((n, m), F32),
            jax.ShapeDtypeStruct((n, m), F32),
        ],
    )(hidden, W1a, W1b)


GSUB = 40
GSCH = 200


def _make_gather(E, M):
    T = E // NW
    n_super = T // GSCH
    mesh = plsc.VectorSubcoreMesh(core_axis_name="c", subcore_axis_name="s")

    @functools.partial(
        pl.kernel,
        mesh=mesh,
        out_type=[
            jax.ShapeDtypeStruct((E, M), F32),
            jax.ShapeDtypeStruct((E, M), F32),
        ],
        scratch_types=[
            pltpu.VMEM((GSCH,), I32),
            pltpu.VMEM((GSCH,), I32),
            pltpu.VMEM((GSCH,), I32),
            pltpu.VMEM((GSCH,), I32),
            pltpu.VMEM((GSCH, M), F32),
            pltpu.VMEM((GSCH, M), F32),
            pltpu.VMEM((GSCH, M), F32),
            pltpu.VMEM((GSCH, M), F32),
            pltpu.SemaphoreType.DMA,
            pltpu.SemaphoreType.DMA,
            pltpu.SemaphoreType.DMA,
            pltpu.SemaphoreType.DMA,
            pltpu.SemaphoreType.DMA,
            pltpu.SemaphoreType.DMA,
        ],
    )
    def gather_kernel(p_hbm, q_hbm, e0_hbm, e1_hbm, ga_hbm, gb_hbm,
                      idx0a, idx0b, idx1a, idx1b, bufa0, bufa1, bufb0, bufb1,
                      semI0, semI1, semG0, semG1, semO0, semO1):
        wid = lax.axis_index("s") * NC + lax.axis_index("c")
        idx0 = (idx0a, idx0b)
        idx1 = (idx1a, idx1b)
        bufa = (bufa0, bufa1)
        bufb = (bufb0, bufb1)
        semI = (semI0, semI1)
        semG = (semG0, semG1)
        semO = (semO0, semO1)

        def fire_idx(j, b):
            base = wid * T + j * GSCH
            pltpu.async_copy(e0_hbm.at[pl.ds(base, GSCH)], idx0[b], semI[b])
            pltpu.async_copy(e1_hbm.at[pl.ds(base, GSCH)], idx1[b], semI[b])

        def drain_idx(b):
            pltpu.make_async_copy(e0_hbm.at[pl.ds(0, GSCH)], idx0[b], semI[b]).wait()
            pltpu.make_async_copy(e1_hbm.at[pl.ds(0, GSCH)], idx1[b], semI[b]).wait()

        def fire_gathers(b):
            for k in range(GSCH // GSUB):
                sl = pl.ds(k * GSUB, GSUB)
                pltpu.async_copy(p_hbm.at[idx0[b].at[sl]], bufa[b].at[sl], semG[b])
                pltpu.async_copy(q_hbm.at[idx1[b].at[sl]], bufb[b].at[sl], semG[b])

        def drain_gathers(b):
            for k in range(GSCH // GSUB):
                sl = pl.ds(k * GSUB, GSUB)
                pltpu.make_async_copy(p_hbm.at[idx0[b].at[sl]], bufa[b].at[sl], semG[b]).wait()
                pltpu.make_async_copy(q_hbm.at[idx1[b].at[sl]], bufb[b].at[sl], semG[b]).wait()

        def fire_out(j, b):
            base = wid * T + j * GSCH
            pltpu.async_copy(bufa[b], ga_hbm.at[pl.ds(base, GSCH)], semO[b])
            pltpu.async_copy(bufb[b], gb_hbm.at[pl.ds(base, GSCH)], semO[b])

        def drain_out(b):
            pltpu.make_async_copy(bufa[b], ga_hbm.at[pl.ds(0, GSCH)], semO[b]).wait()
            pltpu.make_async_copy(bufb[b], gb_hbm.at[pl.ds(0, GSCH)], semO[b]).wait()

        def half(t, j, b, first):
            drain_idx(b)

            @pl.when(t > 0)
            def _():
                drain_out(b)

            fire_gathers(b)
            if first:
                @pl.when(t > 0)
                def _():
                    drain_gathers(1 - b)
                    fire_out(j - 1, 1 - b)
            else:
                drain_gathers(1 - b)
                fire_out(j - 1, 1 - b)
            @pl.when(j + 1 < n_super)
            def _():
                fire_idx(j + 1, 1 - b)

        fire_idx(0, 0)

        def body(t, _):
            half(t, 2 * t, 0, True)
            half(t, 2 * t + 1, 1, False)
            return 0

        lax.fori_loop(0, n_super // 2, body, 0)
        drain_gathers(1)
        fire_out(n_super - 1, 1)
        drain_out(0)
        drain_out(1)

    return gather_kernel


def _make_coords(E, N):
    T = E // NW
    n_super = T // SCH
    nv = SCH // 16
    mesh = plsc.VectorSubcoreMesh(core_axis_name="c", subcore_axis_name="s")

    @functools.partial(
        pl.kernel,
        mesh=mesh,
        out_type=jax.ShapeDtypeStruct((E * 8,), F32),
        compiler_params=pltpu.CompilerParams(needs_layout_passes=False),
        scratch_types=[
            pltpu.VMEM((N,), F32),
            pltpu.VMEM((N,), F32),
            pltpu.VMEM((N,), F32),
            pltpu.VMEM((SCH,), I32),
            pltpu.VMEM((SCH,), I32),
            pltpu.VMEM((SCH * 8,), F32),
            pltpu.SemaphoreType.DMA,
        ],
    )
    def coords_kernel(cx_hbm, cy_hbm, cz_hbm, e0_hbm, e1_hbm, cdn_hbm,
                      cxv, cyv, czv, idx0, idx1, stage, sem):
        wid = lax.axis_index("s") * NC + lax.axis_index("c")
        pltpu.sync_copy(cx_hbm, cxv)
        pltpu.sync_copy(cy_hbm, cyv)
        pltpu.sync_copy(cz_hbm, czv)
        zero16 = jnp.zeros((16,), F32)
        for u in range(SCH * 8 // 16):
            stage[pl.ds(u * 16, 16)] = zero16

        def body(j, _):
            base = wid * T + j * SCH
            pltpu.sync_copy(e0_hbm.at[pl.ds(base, SCH)], idx0)
            pltpu.sync_copy(e1_hbm.at[pl.ds(base, SCH)], idx1)
            for v in range(nv):
                i0 = idx0[pl.ds(v * 16, 16)]
                i1 = idx1[pl.ds(v * 16, 16)]
                dx = plsc.load_gather(cxv, [i0]) - plsc.load_gather(cxv, [i1])
                dy = plsc.load_gather(cyv, [i0]) - plsc.load_gather(cyv, [i1])
                dz = plsc.load_gather(czv, [i0]) - plsc.load_gather(czv, [i1])
                n2 = dx * dx + dy * dy + dz * dz
                rowb = (v * 16 + _iota16()) * 8
                plsc.store_scatter(stage, [rowb], dx)
                plsc.store_scatter(stage, [rowb + 1], dy)
                plsc.store_scatter(stage, [rowb + 2], dz)
                plsc.store_scatter(stage, [rowb + 3], n2)
            pltpu.sync_copy(stage, cdn_hbm.at[pl.ds(base * 8, SCH * 8)])
            return 0

        lax.fori_loop(0, n_super, body, 0)

    return coords_kernel


def _edge_mlp_body(ga_ref, gb_ref, cd_ref,
                   w1c_ref, b1_ref, w2_ref, b2_ref,
                   wc1_ref, bc1_ref, wc2_ref,
                   m_ref, tr_ref):
    cd = cd_ref[...]
    n2 = cd[:, 3:4]
    pre1 = ga_ref[...] + gb_ref[...] + n2 * w1c_ref[...] + b1_ref[...]
    x1 = jax.nn.silu(pre1)
    m = jax.nn.silu(jnp.dot(x1.astype(BF16), w2_ref[...],
                            preferred_element_type=F32) + b2_ref[...])
    y = jax.nn.silu(jnp.dot(m.astype(BF16), wc1_ref[...],
                            preferred_element_type=F32) + bc1_ref[...])
    s = jnp.dot(y, wc2_ref[...], preferred_element_type=F32)
    lane = lax.broadcasted_iota(I32, cd.shape, 1)
    tr_ref[...] = jnp.where(lane == 3, 1.0, cd * s)
    m_ref[...] = m


def _edge_mlp(ga, gb, cd, w1c, b1, W2, b2, Wc1, bc1, Wc2, blk):
    E, M = ga.shape
    full = lambda i: (0, 0)
    return pl.pallas_call(
        _edge_mlp_body,
        grid=(E // blk,),
        in_specs=[
            pl.BlockSpec((blk, M), lambda i: (i, 0)),
            pl.BlockSpec((blk, M), lambda i: (i, 0)),
            pl.BlockSpec((blk, 8), lambda i: (i, 0)),
            pl.BlockSpec((1, M), full),
            pl.BlockSpec((1, M), full),
            pl.BlockSpec((M, M), full),
            pl.BlockSpec((1, M), full),
            pl.BlockSpec((M, M), full),
            pl.BlockSpec((1, M), full),
            pl.BlockSpec((M, 1), full),
        ],
        out_specs=[
            pl.BlockSpec((blk, M), lambda i: (i, 0)),
            pl.BlockSpec((blk, 8), lambda i: (i, 0)),
        ],
        out_shape=[
            jax.ShapeDtypeStruct((E, M), F32),
            jax.ShapeDtypeStruct((E, 8), F32),
        ],
    )(ga, gb, cd, w1c.reshape(1, M), b1.reshape(1, M), W2.astype(BF16),
      b2.reshape(1, M), Wc1.astype(BF16), bc1.reshape(1, M), Wc2)


def _make_scatter(E, NP, M):
    T = E // NW
    n_chunk = T // SUB
    rows_pt = NP // NS
    mesh = plsc.VectorSubcoreMesh(core_axis_name="c", subcore_axis_name="s")

    @functools.partial(
        pl.kernel,
        mesh=mesh,
        out_type=[
            jax.ShapeDtypeStruct((NC, NP, M), F32),
            jax.ShapeDtypeStruct((NC, NP, 128), F32),
        ],
        compiler_params=pltpu.CompilerParams(needs_layout_passes=False),
        scratch_types=[
            pltpu.VMEM((SUB,), I32),
            pltpu.VMEM((SUB,), I32),
            pltpu.VMEM((SUB, M), F32),
            pltpu.VMEM((SUB, M), F32),
            pltpu.VMEM((SUB * 8,), F32),
            pltpu.VMEM((SUB * 8,), F32),
            pltpu.VMEM((SUB, 128), F32),
            pltpu.VMEM((SUB, 128), F32),
            pltpu.VMEM_SHARED((NP, M), F32),
            pltpu.SemaphoreType.DMA,
            pltpu.SemaphoreType.DMA,
            pltpu.SemaphoreType.DMA,
            pltpu.SemaphoreType.DMA,
        ],
    )
    def scatter_kernel(m_hbm, trf_hbm, e0_hbm, zm_hbm,
                       maggp_hbm, caggp_hbm,
                       idxc0, idxc1, mbuf0, mbuf1, tbuf0, tbuf1, stg0, stg1,
                       sh, semL0, semL1, semS0, semS1):
        cid = lax.axis_index("c")
        sid = lax.axis_index("s")
        wid = sid * NC + cid
        r0 = sid * rows_pt
        idxc = (idxc0, idxc1)
        mbuf = (mbuf0, mbuf1)
        tbuf = (tbuf0, tbuf1)
        stg = (stg0, stg1)
        semL = (semL0, semL1)
        semS = (semS0, semS1)

        pltpu.sync_copy(zm_hbm.at[pl.ds(0, SUB)], stg0)
        pltpu.sync_copy(zm_hbm.at[pl.ds(0, SUB)], stg1)
        pltpu.sync_copy(zm_hbm.at[pl.ds(r0, rows_pt)],
                        sh.at[pl.ds(r0, rows_pt)])
        plsc.subcore_barrier()

        def load1(j, b):
            base = wid * T + j * SUB
            pltpu.async_copy(e0_hbm.at[pl.ds(base, SUB)], idxc[b], semL[b])
            pltpu.async_copy(m_hbm.at[pl.ds(base, SUB)], mbuf[b], semL[b])

        def drain_load1(b):
            pltpu.make_async_copy(e0_hbm.at[pl.ds(0, SUB)], idxc[b], semL[b]).wait()
            pltpu.make_async_copy(m_hbm.at[pl.ds(0, SUB)], mbuf[b], semL[b]).wait()

        def fire_stream1(b):
            pltpu.async_copy(mbuf[b], sh.at[idxc[b]], semS[b], add=True)

        def drain_stream1(b):
            pltpu.make_async_copy(mbuf[b], sh.at[idxc[b]], semS[b]).wait()

        def half1(t, j, b, guard):
            if guard:
                drain_stream1(1 - b)
            else:
                @pl.when(t > 0)
                def _():
                    drain_stream1(1 - b)
            load1(j + 1, 1 - b)
            drain_load1(b)
            fire_stream1(b)

        load1(0, 0)

        def body1(t, _):
            half1(t, 2 * t, 0, False)
            half1(t, 2 * t + 1, 1, True)
            return 0

        lax.fori_loop(0, (n_chunk - 1) // 2, body1, 0)
        drain_stream1(1)
        drain_load1(0)
        fire_stream1(0)
        drain_stream1(0)

        plsc.subcore_barrier()
        pltpu.sync_copy(sh.at[pl.ds(r0, rows_pt)],
                        maggp_hbm.at[cid, pl.ds(r0, rows_pt)])
        pltpu.sync_copy(zm_hbm.at[pl.ds(r0, rows_pt)],
                        sh.at[pl.ds(r0, rows_pt)])
        plsc.subcore_barrier()

        iota = _iota16()
        rloc = iota >> 3
        cloc = iota & 7

        def load2(j, b):
            base = wid * T + j * SUB
            pltpu.async_copy(e0_hbm.at[pl.ds(base, SUB)], idxc[b], semL[b])
            pltpu.async_copy(trf_hbm.at[pl.ds(base * 8, SUB * 8)], tbuf[b], semL[b])

        def drain_load2(b):
            pltpu.make_async_copy(e0_hbm.at[pl.ds(0, SUB)], idxc[b], semL[b]).wait()
            pltpu.make_async_copy(trf_hbm.at[pl.ds(0, SUB * 8)], tbuf[b], semL[b]).wait()

        def fill(b):
            for u in range(SUB // 2):
                vals = tbuf[b][pl.ds(u * 16, 16)]
                plsc.store_scatter(stg[b], [2 * u + rloc, cloc], vals)

        def fire_stream2(b):
            pltpu.async_copy(stg[b], sh.at[idxc[b]], semS[b], add=True)

        def drain_stream2(b):
            pltpu.make_async_copy(stg[b], sh.at[idxc[b]], semS[b]).wait()

        def half2(t, j, b, guard):
            drain_load2(b)
            fill(b)
            fire_stream2(b)
            if guard:
                drain_stream2(1 - b)
            else:
                @pl.when(t > 0)
                def _():
                    drain_stream2(1 - b)
            load2(j + 1, 1 - b)

        load2(0, 0)

        def body2(t, _):
            half2(t, 2 * t, 0, False)
            half2(t, 2 * t + 1, 1, True)
            return 0

        lax.fori_loop(0, (n_chunk - 1) // 2, body2, 0)
        drain_load2(0)
        fill(0)
        fire_stream2(0)
        drain_stream2(1)
        drain_stream2(0)

        plsc.subcore_barrier()
        pltpu.sync_copy(sh.at[pl.ds(r0, rows_pt)],
                        caggp_hbm.at[cid, pl.ds(r0, rows_pt)])

    return scatter_kernel


def _node_mlp_body(cp_ref, h_ref, maggp_ref, caggp_ref,
                   wh1a_ref, wh1b_ref, bh1_ref, wh2_ref, bh2_ref,
                   co_ref, ho_ref):
    magg = maggp_ref[0] + maggp_ref[1]
    cagg = caggp_ref[0] + caggp_ref[1]
    counts = jnp.clip(cagg[:, 3:4], 1.0, None)
    co_ref[...] = cp_ref[...] + cagg[:, :8] / counts
    h = jax.nn.silu(jnp.dot(h_ref[...], wh1a_ref[...], preferred_element_type=F32)
                    + jnp.dot(magg, wh1b_ref[...], preferred_element_type=F32)
                    + bh1_ref[...])
    ho_ref[...] = jnp.dot(h, wh2_ref[...], preferred_element_type=F32) + bh2_ref[...]


def _node_mlp(coords_pad, hidden, maggp, caggp, Wh1a, Wh1b, bh1, Wh2, bh2, blk):
    n, hdim = hidden.shape
    m = Wh1a.shape[1]
    NP = maggp.shape[1]
    full = lambda i: (0, 0)
    return pl.pallas_call(
        _node_mlp_body,
        grid=(n // blk,),
        in_specs=[
            pl.BlockSpec((blk, 8), lambda i: (i, 0)),
            pl.BlockSpec((blk, hdim), lambda i: (i, 0)),
            pl.BlockSpec((NC, blk, m), lambda i: (0, i, 0)),
            pl.BlockSpec((NC, blk, 128), lambda i: (0, i, 0)),
            pl.BlockSpec((hdim, m), full),
            pl.BlockSpec((m, m), full),
            pl.BlockSpec((1, m), full),
            pl.BlockSpec((m, hdim), full),
            pl.BlockSpec((1, hdim), full),
        ],
        out_specs=[
            pl.BlockSpec((blk, 8), lambda i: (i, 0)),
            pl.BlockSpec((blk, hdim), lambda i: (i, 0)),
        ],
        out_shape=[
            jax.ShapeDtypeStruct((n, 8), F32),
            jax.ShapeDtypeStruct((n, hdim), F32),
        ],
    )(coords_pad, hidden, maggp, caggp, Wh1a, Wh1b,
      bh1.reshape(1, m), Wh2, bh2.reshape(1, hdim))


def kernel(coords, hidden, edges, W1, b1, W2, b2, Wc1, bc1, Wc2,
           Wh1, bh1, Wh2, bh2):
    N, H = hidden.shape
    E = edges.shape[1]
    M = W2.shape[0]

    e0 = edges[0]
    e1 = edges[1]
    e0r = e0.reshape(E // SUB, SUB)
    coords_pad = jnp.pad(coords, ((0, 0), (0, 5)))
    cx = coords[:, 0]
    cy = coords[:, 1]
    cz = coords[:, 2]

    W1a = W1[:H]
    W1b = W1[H:2 * H]
    w1c = W1[2 * H]
    Wh1a = Wh1[:H]
    Wh1b = Wh1[H:]

    P, Q = _precompute(hidden, W1a, W1b, blk=2000)
    ga, gb = _make_gather(E, M)(P, Q, e0, e1)
    cdn = _make_coords(E, N)(cx, cy, cz, e0, e1)
    cd = cdn.reshape(E, 8)

    m, tr = _edge_mlp(ga, gb, cd, w1c, b1, W2, b2, Wc1, bc1, Wc2, blk=4000)

    NP = ((N + NS * 8 - 1) // (NS * 8)) * NS * 8
    zm = jnp.zeros((NP, M), F32)
    maggp, caggp = _make_scatter(E, NP, M)(m, tr.reshape(E * 8), e0, zm)

    co8, hidden_out = _node_mlp(coords_pad, hidden, maggp, caggp,
                                Wh1a, Wh1b, bh1, Wh2, bh2, blk=2000)
    coords_out = co8[:, :3]
    return (coords_out, hidden_out)


# speedup vs baseline: 5.6784x; 1.0006x over previous
import functools

import jax
import jax.numpy as jnp
from jax import lax
from jax.experimental import pallas as pl
from jax.experimental.pallas import tpu as pltpu
from jax.experimental.pallas import tpu_sc as plsc

F32 = jnp.float32
BF16 = jnp.bfloat16
I32 = jnp.int32

NC = 2
NS = 16
NW = NC * NS

SUB = 80
KSUB = 5
SCH = SUB * KSUB


def _iota16():
    return lax.iota(I32, 16)


def _precompute_body(h_ref, wa_ref, wb_ref, p_ref, q_ref):
    h = h_ref[...]
    p_ref[...] = jnp.dot(h, wa_ref[...], preferred_element_type=F32)
    q_ref[...] = jnp.dot(h, wb_ref[...], preferred_element_type=F32)


def _precompute(hidden, W1a, W1b, blk):
    n, hdim = hidden.shape
    m = W1a.shape[1]
    return pl.pallas_call(
        _precompute_body,
        grid=(n // blk,),
        in_specs=[
            pl.BlockSpec((blk, hdim), lambda i: (i, 0)),
            pl.BlockSpec((hdim, m), lambda i: (0, 0)),
            pl.BlockSpec((hdim, m), lambda i: (0, 0)),
        ],
        out_specs=[
            pl.BlockSpec((blk, m), lambda i: (i, 0)),
            pl.BlockSpec((blk, m), lambda i: (i, 0)),
        ],
        out_shape=[
            jax.ShapeDtypeStruct((n, m), F32),
            jax.ShapeDtypeStruct((n, m), F32),
        ],
    )(hidden, W1a, W1b)


GSUB = 40
GSCH = 200


def _make_gather(E, M):
    T = E // NW
    n_super = T // GSCH
    mesh = plsc.VectorSubcoreMesh(core_axis_name="c", subcore_axis_name="s")

    @functools.partial(
        pl.kernel,
        mesh=mesh,
        out_type=[
            jax.ShapeDtypeStruct((E, M), F32),
            jax.ShapeDtypeStruct((E, M), F32),
        ],
        scratch_types=[
            pltpu.VMEM((GSCH,), I32),
            pltpu.VMEM((GSCH,), I32),
            pltpu.VMEM((GSCH,), I32),
            pltpu.VMEM((GSCH,), I32),
            pltpu.VMEM((GSCH, M), F32),
            pltpu.VMEM((GSCH, M), F32),
            pltpu.VMEM((GSCH, M), F32),
            pltpu.VMEM((GSCH, M), F32),
            pltpu.SemaphoreType.DMA,
            pltpu.SemaphoreType.DMA,
            pltpu.SemaphoreType.DMA,
            pltpu.SemaphoreType.DMA,
            pltpu.SemaphoreType.DMA,
            pltpu.SemaphoreType.DMA,
        ],
    )
    def gather_kernel(p_hbm, q_hbm, e0_hbm, e1_hbm, ga_hbm, gb_hbm,
                      idx0a, idx0b, idx1a, idx1b, bufa0, bufa1, bufb0, bufb1,
                      semI0, semI1, semG0, semG1, semO0, semO1):
        wid = lax.axis_index("s") * NC + lax.axis_index("c")
        idx0 = (idx0a, idx0b)
        idx1 = (idx1a, idx1b)
        bufa = (bufa0, bufa1)
        bufb = (bufb0, bufb1)
        semI = (semI0, semI1)
        semG = (semG0, semG1)
        semO = (semO0, semO1)

        def fire_idx(j, b):
            base = wid * T + j * GSCH
            pltpu.async_copy(e0_hbm.at[pl.ds(base, GSCH)], idx0[b], semI[b])
            pltpu.async_copy(e1_hbm.at[pl.ds(base, GSCH)], idx1[b], semI[b])

        def drain_idx(b):
            pltpu.make_async_copy(e0_hbm.at[pl.ds(0, GSCH)], idx0[b], semI[b]).wait()
            pltpu.make_async_copy(e1_hbm.at[pl.ds(0, GSCH)], idx1[b], semI[b]).wait()

        def fire_gathers(b):
            for k in range(GSCH // GSUB):
                sl = pl.ds(k * GSUB, GSUB)
                pltpu.async_copy(p_hbm.at[idx0[b].at[sl]], bufa[b].at[sl], semG[b])
                pltpu.async_copy(q_hbm.at[idx1[b].at[sl]], bufb[b].at[sl], semG[b])

        def drain_gathers(b):
            for k in range(GSCH // GSUB):
                sl = pl.ds(k * GSUB, GSUB)
                pltpu.make_async_copy(p_hbm.at[idx0[b].at[sl]], bufa[b].at[sl], semG[b]).wait()
                pltpu.make_async_copy(q_hbm.at[idx1[b].at[sl]], bufb[b].at[sl], semG[b]).wait()

        def fire_out(j, b):
            base = wid * T + j * GSCH
            pltpu.async_copy(bufa[b], ga_hbm.at[pl.ds(base, GSCH)], semO[b])
            pltpu.async_copy(bufb[b], gb_hbm.at[pl.ds(base, GSCH)], semO[b])

        def drain_out(b):
            pltpu.make_async_copy(bufa[b], ga_hbm.at[pl.ds(0, GSCH)], semO[b]).wait()
            pltpu.make_async_copy(bufb[b], gb_hbm.at[pl.ds(0, GSCH)], semO[b]).wait()

        def half(t, j, b, first):
            drain_idx(b)

            @pl.when(t > 0)
            def _():
                drain_out(b)

            fire_gathers(b)
            if first:
                @pl.when(t > 0)
                def _():
                    drain_gathers(1 - b)
                    fire_out(j - 1, 1 - b)
            else:
                drain_gathers(1 - b)
                fire_out(j - 1, 1 - b)
            @pl.when(j + 1 < n_super)
            def _():
                fire_idx(j + 1, 1 - b)

        fire_idx(0, 0)

        def body(t, _):
            half(t, 2 * t, 0, True)
            half(t, 2 * t + 1, 1, False)
            return 0

        lax.fori_loop(0, n_super // 2, body, 0)
        drain_gathers(1)
        fire_out(n_super - 1, 1)
        drain_out(0)
        drain_out(1)

    return gather_kernel


def _make_coords(E, N):
    T = E // NW
    n_super = T // SCH
    nv = SCH // 16
    mesh = plsc.VectorSubcoreMesh(core_axis_name="c", subcore_axis_name="s")

    @functools.partial(
        pl.kernel,
        mesh=mesh,
        out_type=jax.ShapeDtypeStruct((E * 8,), F32),
        compiler_params=pltpu.CompilerParams(needs_layout_passes=False),
        scratch_types=[
            pltpu.VMEM((N,), F32),
            pltpu.VMEM((N,), F32),
            pltpu.VMEM((N,), F32),
            pltpu.VMEM((SCH,), I32),
            pltpu.VMEM((SCH,), I32),
            pltpu.VMEM((SCH * 8,), F32),
            pltpu.SemaphoreType.DMA,
        ],
    )
    def coords_kernel(cx_hbm, cy_hbm, cz_hbm, e0_hbm, e1_hbm, cdn_hbm,
                      cxv, cyv, czv, idx0, idx1, stage, sem):
        wid = lax.axis_index("s") * NC + lax.axis_index("c")
        pltpu.sync_copy(cx_hbm, cxv)
        pltpu.sync_copy(cy_hbm, cyv)
        pltpu.sync_copy(cz_hbm, czv)
        zero16 = jnp.zeros((16,), F32)
        for u in range(SCH * 8 // 16):
            stage[pl.ds(u * 16, 16)] = zero16

        def body(j, _):
            base = wid * T + j * SCH
            pltpu.sync_copy(e0_hbm.at[pl.ds(base, SCH)], idx0)
            pltpu.sync_copy(e1_hbm.at[pl.ds(base, SCH)], idx1)
            for v in range(nv):
                i0 = idx0[pl.ds(v * 16, 16)]
                i1 = idx1[pl.ds(v * 16, 16)]
                dx = plsc.load_gather(cxv, [i0]) - plsc.load_gather(cxv, [i1])
                dy = plsc.load_gather(cyv, [i0]) - plsc.load_gather(cyv, [i1])
                dz = plsc.load_gather(czv, [i0]) - plsc.load_gather(czv, [i1])
                n2 = dx * dx + dy * dy + dz * dz
                rowb = (v * 16 + _iota16()) * 8
                plsc.store_scatter(stage, [rowb], dx)
                plsc.store_scatter(stage, [rowb + 1], dy)
                plsc.store_scatter(stage, [rowb + 2], dz)
                plsc.store_scatter(stage, [rowb + 3], n2)
            pltpu.sync_copy(stage, cdn_hbm.at[pl.ds(base * 8, SCH * 8)])
            return 0

        lax.fori_loop(0, n_super, body, 0)

    return coords_kernel


def _edge_mlp_body(ga_ref, gb_ref, cd_ref,
                   w1c_ref, b1_ref, w2_ref, b2_ref,
                   wc1_ref, bc1_ref, wc2_ref,
                   m_ref, tr_ref):
    cd = cd_ref[...]
    n2 = cd[:, 3:4]
    pre1 = ga_ref[...] + gb_ref[...] + n2 * w1c_ref[...] + b1_ref[...]
    x1 = jax.nn.silu(pre1)
    m = jax.nn.silu(jnp.dot(x1.astype(BF16), w2_ref[...],
                            preferred_element_type=F32) + b2_ref[...])
    y = jax.nn.silu(jnp.dot(m.astype(BF16), wc1_ref[...],
                            preferred_element_type=F32) + bc1_ref[...])
    s = jnp.dot(y, wc2_ref[...], preferred_element_type=F32)
    lane = lax.broadcasted_iota(I32, cd.shape, 1)
    tr_ref[...] = jnp.where(lane == 3, 1.0, cd * s)
    m_ref[...] = m


def _edge_mlp(ga, gb, cd, w1c, b1, W2, b2, Wc1, bc1, Wc2, blk):
    E, M = ga.shape
    full = lambda i: (0, 0)
    return pl.pallas_call(
        _edge_mlp_body,
        grid=(E // blk,),
        in_specs=[
            pl.BlockSpec((blk, M), lambda i: (i, 0)),
            pl.BlockSpec((blk, M), lambda i: (i, 0)),
            pl.BlockSpec((blk, 8), lambda i: (i, 0)),
            pl.BlockSpec((1, M), full),
            pl.BlockSpec((1, M), full),
            pl.BlockSpec((M, M), full),
            pl.BlockSpec((1, M), full),
            pl.BlockSpec((M, M), full),
            pl.BlockSpec((1, M), full),
            pl.BlockSpec((M, 1), full),
        ],
        out_specs=[
            pl.BlockSpec((blk, M), lambda i: (i, 0)),
            pl.BlockSpec((blk, 8), lambda i: (i, 0)),
        ],
        out_shape=[
            jax.ShapeDtypeStruct((E, M), F32),
            jax.ShapeDtypeStruct((E, 8), F32),
        ],
    )(ga, gb, cd, w1c.reshape(1, M), b1.reshape(1, M), W2.astype(BF16),
      b2.reshape(1, M), Wc1.astype(BF16), bc1.reshape(1, M), Wc2)


def _make_scatter(E, NP, M):
    T = E // NW
    n_chunk = T // SUB
    rows_pt = NP // NS
    mesh = plsc.VectorSubcoreMesh(core_axis_name="c", subcore_axis_name="s")

    @functools.partial(
        pl.kernel,
        mesh=mesh,
        out_type=[
            jax.ShapeDtypeStruct((NC, NP, M), F32),
            jax.ShapeDtypeStruct((NC, NP, 128), F32),
        ],
        compiler_params=pltpu.CompilerParams(needs_layout_passes=False),
        scratch_types=[
            pltpu.VMEM((SUB,), I32),
            pltpu.VMEM((SUB,), I32),
            pltpu.VMEM((SUB, M), F32),
            pltpu.VMEM((SUB, M), F32),
            pltpu.VMEM((SUB * 8,), F32),
            pltpu.VMEM((SUB * 8,), F32),
            pltpu.VMEM((SUB, 128), F32),
            pltpu.VMEM((SUB, 128), F32),
            pltpu.VMEM_SHARED((NP, M), F32),
            pltpu.SemaphoreType.DMA,
            pltpu.SemaphoreType.DMA,
            pltpu.SemaphoreType.DMA,
            pltpu.SemaphoreType.DMA,
        ],
    )
    def scatter_kernel(m_hbm, trf_hbm, e0_hbm, zm_hbm,
                       maggp_hbm, caggp_hbm,
                       idxc0, idxc1, mbuf0, mbuf1, tbuf0, tbuf1, stg0, stg1,
                       sh, semL0, semL1, semS0, semS1):
        cid = lax.axis_index("c")
        sid = lax.axis_index("s")
        wid = sid * NC + cid
        r0 = sid * rows_pt
        idxc = (idxc0, idxc1)
        mbuf = (mbuf0, mbuf1)
        tbuf = (tbuf0, tbuf1)
        stg = (stg0, stg1)
        semL = (semL0, semL1)
        semS = (semS0, semS1)

        pltpu.sync_copy(zm_hbm.at[pl.ds(0, SUB)], stg0)
        pltpu.sync_copy(zm_hbm.at[pl.ds(0, SUB)], stg1)
        pltpu.sync_copy(zm_hbm.at[pl.ds(r0, rows_pt)],
                        sh.at[pl.ds(r0, rows_pt)])
        plsc.subcore_barrier()

        def load1(j, b):
            base = wid * T + j * SUB
            pltpu.async_copy(e0_hbm.at[pl.ds(base, SUB)], idxc[b], semL[b])
            pltpu.async_copy(m_hbm.at[pl.ds(base, SUB)], mbuf[b], semL[b])

        def drain_load1(b):
            pltpu.make_async_copy(e0_hbm.at[pl.ds(0, SUB)], idxc[b], semL[b]).wait()
            pltpu.make_async_copy(m_hbm.at[pl.ds(0, SUB)], mbuf[b], semL[b]).wait()

        def fire_stream1(b):
            pltpu.async_copy(mbuf[b], sh.at[idxc[b]], semS[b], add=True)

        def drain_stream1(b):
            pltpu.make_async_copy(mbuf[b], sh.at[idxc[b]], semS[b]).wait()

        def half1(t, j, b, guard):
            if guard:
                drain_stream1(1 - b)
            else:
                @pl.when(t > 0)
                def _():
                    drain_stream1(1 - b)
            load1(j + 1, 1 - b)
            drain_load1(b)
            fire_stream1(b)

        load1(0, 0)

        def body1(t, _):
            half1(t, 2 * t, 0, False)
            half1(t, 2 * t + 1, 1, True)
            return 0

        lax.fori_loop(0, (n_chunk - 1) // 2, body1, 0)
        drain_stream1(1)
        drain_load1(0)
        fire_stream1(0)
        drain_stream1(0)

        plsc.subcore_barrier()
        pltpu.sync_copy(sh.at[pl.ds(r0, rows_pt)],
                        maggp_hbm.at[cid, pl.ds(r0, rows_pt)])
        pltpu.sync_copy(zm_hbm.at[pl.ds(r0, rows_pt)],
                        sh.at[pl.ds(r0, rows_pt)])
        plsc.subcore_barrier()

        iota = _iota16()
        rloc = iota >> 3
        cloc = iota & 7

        def load2(j, b):
            base = wid * T + j * SUB
            pltpu.async_copy(e0_hbm.at[pl.ds(base, SUB)], idxc[b], semL[b])
            pltpu.async_copy(trf_hbm.at[pl.ds(base * 8, SUB * 8)], tbuf[b], semL[b])

        def drain_load2(b):
            pltpu.make_async_copy(e0_hbm.at[pl.ds(0, SUB)], idxc[b], semL[b]).wait()
            pltpu.make_async_copy(trf_hbm.at[pl.ds(0, SUB * 8)], tbuf[b], semL[b]).wait()

        def fill(b):
            for u in range(SUB // 2):
                vals = tbuf[b][pl.ds(u * 16, 16)]
                plsc.store_scatter(stg[b], [2 * u + rloc, cloc], vals)

        def fire_stream2(b):
            pltpu.async_copy(stg[b], sh.at[idxc[b]], semS[b], add=True)

        def drain_stream2(b):
            pltpu.make_async_copy(stg[b], sh.at[idxc[b]], semS[b]).wait()

        def half2(t, j, b, guard):
            drain_load2(b)
            fill(b)
            fire_stream2(b)
            if guard:
                drain_stream2(1 - b)
            else:
                @pl.when(t > 0)
                def _():
                    drain_stream2(1 - b)
            load2(j + 1, 1 - b)

        load2(0, 0)

        def body2(t, _):
            half2(t, 2 * t, 0, False)
            half2(t, 2 * t + 1, 1, True)
            return 0

        lax.fori_loop(0, (n_chunk - 1) // 2, body2, 0)
        drain_load2(0)
        fill(0)
        fire_stream2(0)
        drain_stream2(1)
        drain_stream2(0)

        plsc.subcore_barrier()
        pltpu.sync_copy(sh.at[pl.ds(r0, rows_pt)],
                        caggp_hbm.at[cid, pl.ds(r0, rows_pt)])

    return scatter_kernel


def _node_mlp_body(cp_ref, h_ref, maggp_ref, caggp_ref,
                   wh1a_ref, wh1b_ref, bh1_ref, wh2_ref, bh2_ref,
                   co_ref, ho_ref):
    magg = maggp_ref[0] + maggp_ref[1]
    cagg = caggp_ref[0] + caggp_ref[1]
    counts = jnp.clip(cagg[:, 3:4], 1.0, None)
    co_ref[...] = cp_ref[...] + cagg[:, :8] / counts
    h = jax.nn.silu(jnp.dot(h_ref[...], wh1a_ref[...], preferred_element_type=F32)
                    + jnp.dot(magg, wh1b_ref[...], preferred_element_type=F32)
                    + bh1_ref[...])
    ho_ref[...] = jnp.dot(h, wh2_ref[...], preferred_element_type=F32) + bh2_ref[...]


def _node_mlp(coords_pad, hidden, maggp, caggp, Wh1a, Wh1b, bh1, Wh2, bh2, blk):
    n, hdim = hidden.shape
    m = Wh1a.shape[1]
    NP = maggp.shape[1]
    full = lambda i: (0, 0)
    return pl.pallas_call(
        _node_mlp_body,
        grid=(n // blk,),
        in_specs=[
            pl.BlockSpec((blk, 8), lambda i: (i, 0)),
            pl.BlockSpec((blk, hdim), lambda i: (i, 0)),
            pl.BlockSpec((NC, blk, m), lambda i: (0, i, 0)),
            pl.BlockSpec((NC, blk, 128), lambda i: (0, i, 0)),
            pl.BlockSpec((hdim, m), full),
            pl.BlockSpec((m, m), full),
            pl.BlockSpec((1, m), full),
            pl.BlockSpec((m, hdim), full),
            pl.BlockSpec((1, hdim), full),
        ],
        out_specs=[
            pl.BlockSpec((blk, 8), lambda i: (i, 0)),
            pl.BlockSpec((blk, hdim), lambda i: (i, 0)),
        ],
        out_shape=[
            jax.ShapeDtypeStruct((n, 8), F32),
            jax.ShapeDtypeStruct((n, hdim), F32),
        ],
    )(coords_pad, hidden, maggp, caggp, Wh1a, Wh1b,
      bh1.reshape(1, m), Wh2, bh2.reshape(1, hdim))


def kernel(coords, hidden, edges, W1, b1, W2, b2, Wc1, bc1, Wc2,
           Wh1, bh1, Wh2, bh2):
    N, H = hidden.shape
    E = edges.shape[1]
    M = W2.shape[0]

    e0 = edges[0]
    e1 = edges[1]
    coords_pad = jnp.pad(coords, ((0, 0), (0, 5)))
    cx = coords[:, 0]
    cy = coords[:, 1]
    cz = coords[:, 2]

    W1a = W1[:H]
    W1b = W1[H:2 * H]
    w1c = W1[2 * H]
    Wh1a = Wh1[:H]
    Wh1b = Wh1[H:]

    P, Q = _precompute(hidden, W1a, W1b, blk=2000)
    ga, gb = _make_gather(E, M)(P, Q, e0, e1)
    cdn = _make_coords(E, N)(cx, cy, cz, e0, e1)
    cd = cdn.reshape(E, 8)

    m, tr = _edge_mlp(ga, gb, cd, w1c, b1, W2, b2, Wc1, bc1, Wc2, blk=4000)

    NP = ((N + NS * 8 - 1) // (NS * 8)) * NS * 8
    zm = jnp.zeros((NP, M), F32)
    maggp, caggp = _make_scatter(E, NP, M)(m, tr.reshape(E * 8), e0, zm)

    co8, hidden_out = _node_mlp(coords_pad, hidden, maggp, caggp,
                                Wh1a, Wh1b, bh1, Wh2, bh2, blk=2000)
    coords_out = co8[:, :3]
    return (coords_out, hidden_out)


# speedup vs baseline: 5.7187x; 1.0071x over previous
import functools

import jax
import jax.numpy as jnp
from jax import lax
from jax.experimental import pallas as pl
from jax.experimental.pallas import tpu as pltpu
from jax.experimental.pallas import tpu_sc as plsc

F32 = jnp.float32
BF16 = jnp.bfloat16
I32 = jnp.int32

NC = 2
NS = 16
NW = NC * NS

SUB = 80
KSUB = 5
SCH = SUB * KSUB


def _iota16():
    return lax.iota(I32, 16)


def _precompute_body(h_ref, wa_ref, wb_ref, p_ref, q_ref):
    h = h_ref[...]
    p_ref[...] = jnp.dot(h, wa_ref[...], preferred_element_type=F32)
    q_ref[...] = jnp.dot(h, wb_ref[...], preferred_element_type=F32)


def _precompute(hidden, W1a, W1b, blk):
    n, hdim = hidden.shape
    m = W1a.shape[1]
    return pl.pallas_call(
        _precompute_body,
        grid=(n // blk,),
        in_specs=[
            pl.BlockSpec((blk, hdim), lambda i: (i, 0)),
            pl.BlockSpec((hdim, m), lambda i: (0, 0)),
            pl.BlockSpec((hdim, m), lambda i: (0, 0)),
        ],
        out_specs=[
            pl.BlockSpec((blk, m), lambda i: (i, 0)),
            pl.BlockSpec((blk, m), lambda i: (i, 0)),
        ],
        out_shape=[
            jax.ShapeDtypeStruct((n, m), F32),
            jax.ShapeDtypeStruct((n, m), F32),
        ],
    )(hidden, W1a, W1b)


GSUB = 40
GSCH = 200


def _make_gather(E, M):
    T = E // NW
    n_super = T // GSCH
    mesh = plsc.VectorSubcoreMesh(core_axis_name="c", subcore_axis_name="s")

    @functools.partial(
        pl.kernel,
        mesh=mesh,
        out_type=[
            jax.ShapeDtypeStruct((E, M), F32),
            jax.ShapeDtypeStruct((E, M), F32),
        ],
        scratch_types=[
            pltpu.VMEM((GSCH,), I32),
            pltpu.VMEM((GSCH,), I32),
            pltpu.VMEM((GSCH,), I32),
            pltpu.VMEM((GSCH,), I32),
            pltpu.VMEM((GSCH, M), F32),
            pltpu.VMEM((GSCH, M), F32),
            pltpu.VMEM((GSCH, M), F32),
            pltpu.VMEM((GSCH, M), F32),
            pltpu.SemaphoreType.DMA,
            pltpu.SemaphoreType.DMA,
            pltpu.SemaphoreType.DMA,
            pltpu.SemaphoreType.DMA,
            pltpu.SemaphoreType.DMA,
            pltpu.SemaphoreType.DMA,
        ],
    )
    def gather_kernel(p_hbm, q_hbm, e0_hbm, e1_hbm, ga_hbm, gb_hbm,
                      idx0a, idx0b, idx1a, idx1b, bufa0, bufa1, bufb0, bufb1,
                      semI0, semI1, semG0, semG1, semO0, semO1):
        wid = lax.axis_index("s") * NC + lax.axis_index("c")
        idx0 = (idx0a, idx0b)
        idx1 = (idx1a, idx1b)
        bufa = (bufa0, bufa1)
        bufb = (bufb0, bufb1)
        semI = (semI0, semI1)
        semG = (semG0, semG1)
        semO = (semO0, semO1)

        def fire_idx(j, b):
            base = wid * T + j * GSCH
            pltpu.async_copy(e0_hbm.at[pl.ds(base, GSCH)], idx0[b], semI[b])
            pltpu.async_copy(e1_hbm.at[pl.ds(base, GSCH)], idx1[b], semI[b])

        def drain_idx(b):
            pltpu.make_async_copy(e0_hbm.at[pl.ds(0, GSCH)], idx0[b], semI[b]).wait()
            pltpu.make_async_copy(e1_hbm.at[pl.ds(0, GSCH)], idx1[b], semI[b]).wait()

        def fire_gathers(b):
            for k in range(GSCH // GSUB):
                sl = pl.ds(k * GSUB, GSUB)
                pltpu.async_copy(p_hbm.at[idx0[b].at[sl]], bufa[b].at[sl], semG[b])
                pltpu.async_copy(q_hbm.at[idx1[b].at[sl]], bufb[b].at[sl], semG[b])

        def drain_gathers(b):
            for k in range(GSCH // GSUB):
                sl = pl.ds(k * GSUB, GSUB)
                pltpu.make_async_copy(p_hbm.at[idx0[b].at[sl]], bufa[b].at[sl], semG[b]).wait()
                pltpu.make_async_copy(q_hbm.at[idx1[b].at[sl]], bufb[b].at[sl], semG[b]).wait()

        def fire_out(j, b):
            base = wid * T + j * GSCH
            pltpu.async_copy(bufa[b], ga_hbm.at[pl.ds(base, GSCH)], semO[b])
            pltpu.async_copy(bufb[b], gb_hbm.at[pl.ds(base, GSCH)], semO[b])

        def drain_out(b):
            pltpu.make_async_copy(bufa[b], ga_hbm.at[pl.ds(0, GSCH)], semO[b]).wait()
            pltpu.make_async_copy(bufb[b], gb_hbm.at[pl.ds(0, GSCH)], semO[b]).wait()

        def half(t, j, b, first):
            drain_idx(b)

            @pl.when(t > 0)
            def _():
                drain_out(b)

            fire_gathers(b)
            if first:
                @pl.when(t > 0)
                def _():
                    drain_gathers(1 - b)
                    fire_out(j - 1, 1 - b)
            else:
                drain_gathers(1 - b)
                fire_out(j - 1, 1 - b)
            @pl.when(j + 1 < n_super)
            def _():
                fire_idx(j + 1, 1 - b)

        fire_idx(0, 0)

        def body(t, _):
            half(t, 2 * t, 0, True)
            half(t, 2 * t + 1, 1, False)
            return 0

        lax.fori_loop(0, n_super // 2, body, 0)
        drain_gathers(1)
        fire_out(n_super - 1, 1)
        drain_out(0)
        drain_out(1)

    return gather_kernel


def _make_coords(E, N):
    T = E // NW
    n_super = T // SCH
    nv = SCH // 16
    mesh = plsc.VectorSubcoreMesh(core_axis_name="c", subcore_axis_name="s")

    @functools.partial(
        pl.kernel,
        mesh=mesh,
        out_type=jax.ShapeDtypeStruct((E * 8,), F32),
        compiler_params=pltpu.CompilerParams(needs_layout_passes=False),
        scratch_types=[
            pltpu.VMEM((N,), F32),
            pltpu.VMEM((N,), F32),
            pltpu.VMEM((N,), F32),
            pltpu.VMEM((SCH,), I32),
            pltpu.VMEM((SCH,), I32),
            pltpu.VMEM((SCH * 8,), F32),
            pltpu.SemaphoreType.DMA,
        ],
    )
    def coords_kernel(cx_hbm, cy_hbm, cz_hbm, e0_hbm, e1_hbm, cdn_hbm,
                      cxv, cyv, czv, idx0, idx1, stage, sem):
        wid = lax.axis_index("s") * NC + lax.axis_index("c")
        pltpu.sync_copy(cx_hbm, cxv)
        pltpu.sync_copy(cy_hbm, cyv)
        pltpu.sync_copy(cz_hbm, czv)
        zero16 = jnp.zeros((16,), F32)
        for u in range(SCH * 8 // 16):
            stage[pl.ds(u * 16, 16)] = zero16

        def body(j, _):
            base = wid * T + j * SCH
            pltpu.sync_copy(e0_hbm.at[pl.ds(base, SCH)], idx0)
            pltpu.sync_copy(e1_hbm.at[pl.ds(base, SCH)], idx1)
            for v in range(nv):
                i0 = idx0[pl.ds(v * 16, 16)]
                i1 = idx1[pl.ds(v * 16, 16)]
                dx = plsc.load_gather(cxv, [i0]) - plsc.load_gather(cxv, [i1])
                dy = plsc.load_gather(cyv, [i0]) - plsc.load_gather(cyv, [i1])
                dz = plsc.load_gather(czv, [i0]) - plsc.load_gather(czv, [i1])
                n2 = dx * dx + dy * dy + dz * dz
                rowb = (v * 16 + _iota16()) * 8
                plsc.store_scatter(stage, [rowb], dx)
                plsc.store_scatter(stage, [rowb + 1], dy)
                plsc.store_scatter(stage, [rowb + 2], dz)
                plsc.store_scatter(stage, [rowb + 3], n2)
            pltpu.sync_copy(stage, cdn_hbm.at[pl.ds(base * 8, SCH * 8)])
            return 0

        lax.fori_loop(0, n_super, body, 0)

    return coords_kernel


def _edge_mlp_body(ga_ref, gb_ref, cd_ref,
                   w1c_ref, b1_ref, w2_ref, b2_ref,
                   wc1_ref, bc1_ref, wc2_ref,
                   m_ref, tr_ref):
    cd = cd_ref[...]
    n2 = cd[:, 3:4]
    pre1 = ga_ref[...] + gb_ref[...] + n2 * w1c_ref[...] + b1_ref[...]
    x1 = jax.nn.silu(pre1)
    m = jax.nn.silu(jnp.dot(x1.astype(BF16), w2_ref[...],
                            preferred_element_type=F32) + b2_ref[...])
    y = jax.nn.silu(jnp.dot(m.astype(BF16), wc1_ref[...],
                            preferred_element_type=F32) + bc1_ref[...])
    s = jnp.dot(y, wc2_ref[...], preferred_element_type=F32)
    lane = lax.broadcasted_iota(I32, cd.shape, 1)
    tr_ref[...] = jnp.where(lane == 3, 1.0, cd * s)
    m_ref[...] = m


def _edge_mlp(ga, gb, cd, w1c, b1, W2, b2, Wc1, bc1, Wc2, blk):
    E, M = ga.shape
    full = lambda i: (0, 0)
    return pl.pallas_call(
        _edge_mlp_body,
        grid=(E // blk,),
        in_specs=[
            pl.BlockSpec((blk, M), lambda i: (i, 0)),
            pl.BlockSpec((blk, M), lambda i: (i, 0)),
            pl.BlockSpec((blk, 8), lambda i: (i, 0)),
            pl.BlockSpec((1, M), full),
            pl.BlockSpec((1, M), full),
            pl.BlockSpec((M, M), full),
            pl.BlockSpec((1, M), full),
            pl.BlockSpec((M, M), full),
            pl.BlockSpec((1, M), full),
            pl.BlockSpec((M, 1), full),
        ],
        out_specs=[
            pl.BlockSpec((blk, M), lambda i: (i, 0)),
            pl.BlockSpec((blk, 8), lambda i: (i, 0)),
        ],
        out_shape=[
            jax.ShapeDtypeStruct((E, M), F32),
            jax.ShapeDtypeStruct((E, 8), F32),
        ],
    )(ga, gb, cd, w1c.reshape(1, M), b1.reshape(1, M), W2.astype(BF16),
      b2.reshape(1, M), Wc1.astype(BF16), bc1.reshape(1, M), Wc2)


def _make_scatter(E1, E2, NP, M):
    T1 = E1 // NW
    T2 = E2 // NW
    n1 = T1 // SUB
    n2 = T2 // SUB
    rows_pt = NP // NS
    mesh = plsc.VectorSubcoreMesh(core_axis_name="c", subcore_axis_name="s")

    @functools.partial(
        pl.kernel,
        mesh=mesh,
        out_type=[
            jax.ShapeDtypeStruct((NC, NP, M), F32),
            jax.ShapeDtypeStruct((NC, NP, 128), F32),
        ],
        compiler_params=pltpu.CompilerParams(needs_layout_passes=False),
        scratch_types=[
            pltpu.VMEM((SUB,), I32),
            pltpu.VMEM((SUB,), I32),
            pltpu.VMEM((SUB, M), F32),
            pltpu.VMEM((SUB, M), F32),
            pltpu.VMEM((SUB * 8,), F32),
            pltpu.VMEM((SUB * 8,), F32),
            pltpu.VMEM((SUB, 128), F32),
            pltpu.VMEM((SUB, 128), F32),
            pltpu.VMEM_SHARED((NP, M), F32),
            pltpu.SemaphoreType.DMA,
            pltpu.SemaphoreType.DMA,
            pltpu.SemaphoreType.DMA,
            pltpu.SemaphoreType.DMA,
        ],
    )
    def scatter_kernel(m1_hbm, trf1_hbm, e0a_hbm, m2_hbm, trf2_hbm, e0b_hbm,
                       zm_hbm, maggp_hbm, caggp_hbm,
                       idxc0, idxc1, mbuf0, mbuf1, tbuf0, tbuf1, stg0, stg1,
                       sh, semL0, semL1, semS0, semS1):
        cid = lax.axis_index("c")
        sid = lax.axis_index("s")
        wid = sid * NC + cid
        r0 = sid * rows_pt
        idxc = (idxc0, idxc1)
        mbuf = (mbuf0, mbuf1)
        tbuf = (tbuf0, tbuf1)
        stg = (stg0, stg1)
        semL = (semL0, semL1)
        semS = (semS0, semS1)

        pltpu.sync_copy(zm_hbm.at[pl.ds(0, SUB)], stg0)
        pltpu.sync_copy(zm_hbm.at[pl.ds(0, SUB)], stg1)
        pltpu.sync_copy(zm_hbm.at[pl.ds(r0, rows_pt)],
                        sh.at[pl.ds(r0, rows_pt)])
        plsc.subcore_barrier()

        def run_ring(n_chunk, load, drain_load, fire, drain_fire):
            load(0, 0)

            def half(t, j, b, first):
                if first:
                    @pl.when(t > 0)
                    def _():
                        drain_fire(1 - b)
                else:
                    drain_fire(1 - b)

                @pl.when(j + 1 < n_chunk)
                def _():
                    load(j + 1, 1 - b)

                drain_load(b)
                fire(b)

            def body(t, _):
                half(t, 2 * t, 0, True)
                half(t, 2 * t + 1, 1, False)
                return 0

            lax.fori_loop(0, n_chunk // 2, body, 0)
            if n_chunk % 2 == 1:
                drain_fire(1)
                drain_load(0)
                fire(0)
                drain_fire(0)
            else:
                drain_fire(1)

        def phase1(m_hbm, e0_hbm, T, n_chunk):
            def load(j, b):
                base = wid * T + j * SUB
                pltpu.async_copy(e0_hbm.at[pl.ds(base, SUB)], idxc[b], semL[b])
                pltpu.async_copy(m_hbm.at[pl.ds(base, SUB)], mbuf[b], semL[b])

            def drain_load(b):
                pltpu.make_async_copy(e0_hbm.at[pl.ds(0, SUB)], idxc[b], semL[b]).wait()
                pltpu.make_async_copy(m_hbm.at[pl.ds(0, SUB)], mbuf[b], semL[b]).wait()

            def fire(b):
                pltpu.async_copy(mbuf[b], sh.at[idxc[b]], semS[b], add=True)

            def drain_fire(b):
                pltpu.make_async_copy(mbuf[b], sh.at[idxc[b]], semS[b]).wait()

            run_ring(n_chunk, load, drain_load, fire, drain_fire)

        phase1(m1_hbm, e0a_hbm, T1, n1)
        phase1(m2_hbm, e0b_hbm, T2, n2)

        plsc.subcore_barrier()
        pltpu.sync_copy(sh.at[pl.ds(r0, rows_pt)],
                        maggp_hbm.at[cid, pl.ds(r0, rows_pt)])
        pltpu.sync_copy(zm_hbm.at[pl.ds(r0, rows_pt)],
                        sh.at[pl.ds(r0, rows_pt)])
        plsc.subcore_barrier()

        iota = _iota16()
        rloc = iota >> 3
        cloc = iota & 7

        def phase2(trf_hbm, e0_hbm, T, n_chunk):
            def load(j, b):
                base = wid * T + j * SUB
                pltpu.async_copy(e0_hbm.at[pl.ds(base, SUB)], idxc[b], semL[b])
                pltpu.async_copy(trf_hbm.at[pl.ds(base * 8, SUB * 8)], tbuf[b], semL[b])

            def drain_load(b):
                pltpu.make_async_copy(e0_hbm.at[pl.ds(0, SUB)], idxc[b], semL[b]).wait()
                pltpu.make_async_copy(trf_hbm.at[pl.ds(0, SUB * 8)], tbuf[b], semL[b]).wait()

            def fire(b):
                for u in range(SUB // 2):
                    vals = tbuf[b][pl.ds(u * 16, 16)]
                    plsc.store_scatter(stg[b], [2 * u + rloc, cloc], vals)
                pltpu.async_copy(stg[b], sh.at[idxc[b]], semS[b], add=True)

            def drain_fire(b):
                pltpu.make_async_copy(stg[b], sh.at[idxc[b]], semS[b]).wait()

            run_ring(n_chunk, load, drain_load, fire, drain_fire)

        phase2(trf1_hbm, e0a_hbm, T1, n1)
        phase2(trf2_hbm, e0b_hbm, T2, n2)

        plsc.subcore_barrier()
        pltpu.sync_copy(sh.at[pl.ds(r0, rows_pt)],
                        caggp_hbm.at[cid, pl.ds(r0, rows_pt)])

    return scatter_kernel


def _node_mlp_body(cp_ref, h_ref, maggp_ref, caggp_ref,
                   wh1a_ref, wh1b_ref, bh1_ref, wh2_ref, bh2_ref,
                   co_ref, ho_ref):
    magg = maggp_ref[0] + maggp_ref[1]
    cagg = caggp_ref[0] + caggp_ref[1]
    counts = jnp.clip(cagg[:, 3:4], 1.0, None)
    co_ref[...] = cp_ref[...] + cagg[:, :8] / counts
    h = jax.nn.silu(jnp.dot(h_ref[...], wh1a_ref[...], preferred_element_type=F32)
                    + jnp.dot(magg, wh1b_ref[...], preferred_element_type=F32)
                    + bh1_ref[...])
    ho_ref[...] = jnp.dot(h, wh2_ref[...], preferred_element_type=F32) + bh2_ref[...]


def _node_mlp(coords_pad, hidden, maggp, caggp, Wh1a, Wh1b, bh1, Wh2, bh2, blk):
    n, hdim = hidden.shape
    m = Wh1a.shape[1]
    NP = maggp.shape[1]
    full = lambda i: (0, 0)
    return pl.pallas_call(
        _node_mlp_body,
        grid=(n // blk,),
        in_specs=[
            pl.BlockSpec((blk, 8), lambda i: (i, 0)),
            pl.BlockSpec((blk, hdim), lambda i: (i, 0)),
            pl.BlockSpec((NC, blk, m), lambda i: (0, i, 0)),
            pl.BlockSpec((NC, blk, 128), lambda i: (0, i, 0)),
            pl.BlockSpec((hdim, m), full),
            pl.BlockSpec((m, m), full),
            pl.BlockSpec((1, m), full),
            pl.BlockSpec((m, hdim), full),
            pl.BlockSpec((1, hdim), full),
        ],
        out_specs=[
            pl.BlockSpec((blk, 8), lambda i: (i, 0)),
            pl.BlockSpec((blk, hdim), lambda i: (i, 0)),
        ],
        out_shape=[
            jax.ShapeDtypeStruct((n, 8), F32),
            jax.ShapeDtypeStruct((n, hdim), F32),
        ],
    )(coords_pad, hidden, maggp, caggp, Wh1a, Wh1b,
      bh1.reshape(1, m), Wh2, bh2.reshape(1, hdim))


def kernel(coords, hidden, edges, W1, b1, W2, b2, Wc1, bc1, Wc2,
           Wh1, bh1, Wh2, bh2):
    N, H = hidden.shape
    E = edges.shape[1]
    M = W2.shape[0]

    e0 = edges[0]
    e1 = edges[1]
    coords_pad = jnp.pad(coords, ((0, 0), (0, 5)))
    cx = coords[:, 0]
    cy = coords[:, 1]
    cz = coords[:, 2]

    W1a = W1[:H]
    W1b = W1[H:2 * H]
    w1c = W1[2 * H]
    Wh1a = Wh1[:H]
    Wh1b = Wh1[H:]

    P, Q = _precompute(hidden, W1a, W1b, blk=2000)

    E1 = (E * 3) // 5
    E2 = E - E1
    e0a, e0b = e0[:E1], e0[E1:]
    e1a, e1b = e1[:E1], e1[E1:]

    ga1, gb1 = _make_gather(E1, M)(P, Q, e0a, e1a)
    cdn1 = _make_coords(E1, N)(cx, cy, cz, e0a, e1a)
    ga2, gb2 = _make_gather(E2, M)(P, Q, e0b, e1b)
    cdn2 = _make_coords(E2, N)(cx, cy, cz, e0b, e1b)

    m1, tr1 = _edge_mlp(ga1, gb1, cdn1.reshape(E1, 8),
                        w1c, b1, W2, b2, Wc1, bc1, Wc2, blk=4000)
    m2, tr2 = _edge_mlp(ga2, gb2, cdn2.reshape(E2, 8),
                        w1c, b1, W2, b2, Wc1, bc1, Wc2, blk=4000)

    NP = ((N + NS * 8 - 1) // (NS * 8)) * NS * 8
    zm = jnp.zeros((NP, M), F32)
    maggp, caggp = _make_scatter(E1, E2, NP, M)(
        m1, tr1.reshape(E1 * 8), e0a, m2, tr2.reshape(E2 * 8), e0b, zm)

    co8, hidden_out = _node_mlp(coords_pad, hidden, maggp, caggp,
                                Wh1a, Wh1b, bh1, Wh2, bh2, blk=2000)
    coords_out = co8[:, :3]
    return (coords_out, hidden_out)
